# Initial kernel scaffold; baseline (speedup 1.0000x reference)
import jax
import jax.numpy as jnp
from jax.experimental import pallas as pl


def kernel(x, edge_index, edge_attr, msg_W1, msg_b1, msg_W2, msg_b2, soft_W, soft_b, upd_W1, upd_b1, upd_W2, upd_b2):
    raise NotImplementedError("write your pallas kernel here")


# speedup vs baseline: 2.5100x; 2.5100x over previous
import functools

import jax
import jax.numpy as jnp
from jax import lax
from jax.experimental import pallas as pl
from jax.experimental.pallas import tpu as pltpu
from jax.experimental.pallas import tpu_sc as plsc

F32 = jnp.float32

_NC = 2
_NS = 16
_NW = _NC * _NS


def _proj_body(x_ref, wa_ref, wb_ref, pa_ref, pb_ref):
    x = x_ref[...]
    pa_ref[...] = jnp.dot(x, wa_ref[...], preferred_element_type=F32)
    pb_ref[...] = jnp.dot(x, wb_ref[...], preferred_element_type=F32)


def _node_proj(x, w1a, w1b, bn):
    n, h = x.shape
    grid = n // bn
    return pl.pallas_call(
        _proj_body,
        grid=(grid,),
        in_specs=[
            pl.BlockSpec((bn, h), lambda i: (i, 0)),
            pl.BlockSpec((h, h), lambda i: (0, 0)),
            pl.BlockSpec((h, h), lambda i: (0, 0)),
        ],
        out_specs=[
            pl.BlockSpec((bn, h), lambda i: (i, 0)),
            pl.BlockSpec((bn, h), lambda i: (i, 0)),
        ],
        out_shape=[
            jax.ShapeDtypeStruct((n, h), F32),
            jax.ShapeDtypeStruct((n, h), F32),
        ],
    )(x, w1a, w1b)


def _make_gather(n, h, e, chunk):
    ew = e // _NW
    n_chunks = ew // chunk
    mesh = plsc.VectorSubcoreMesh(core_axis_name="c", subcore_axis_name="s")

    @functools.partial(
        pl.kernel,
        out_type=(
            jax.ShapeDtypeStruct((e, h), F32),
            jax.ShapeDtypeStruct((e, h), F32),
        ),
        mesh=mesh,
        scratch_types=[
            pltpu.VMEM((chunk,), jnp.int32),
            pltpu.VMEM((chunk,), jnp.int32),
            pltpu.VMEM((chunk, h), F32),
            pltpu.VMEM((chunk, h), F32),
            pltpu.SemaphoreType.DMA,
            pltpu.SemaphoreType.DMA,
        ],
    )
    def gather_k(pa_hbm, pb_hbm, src_hbm, dst_hbm, ga_hbm, gb_hbm,
                 idx_a, idx_b, rows_a, rows_b, sem_a, sem_b):
        wid = lax.axis_index("s") * _NC + lax.axis_index("c")
        base = wid * ew

        def body(i, carry):
            off = base + i * chunk
            pltpu.sync_copy(src_hbm.at[pl.ds(off, chunk)], idx_a)
            pltpu.sync_copy(dst_hbm.at[pl.ds(off, chunk)], idx_b)
            cp_a = pltpu.async_copy(pa_hbm.at[idx_a], rows_a, sem_a)
            cp_b = pltpu.async_copy(pb_hbm.at[idx_b], rows_b, sem_b)
            cp_a.wait()
            cp_b.wait()
            pltpu.sync_copy(rows_a, ga_hbm.at[pl.ds(off, chunk)])
            pltpu.sync_copy(rows_b, gb_hbm.at[pl.ds(off, chunk)])
            return carry

        lax.fori_loop(0, n_chunks, body, 0)

    return gather_k


def _edge_body(ga_ref, gb_ref, ea_ref, w1e_ref, b1_ref, w2_ref, b2_ref,
               swr_ref, sb_ref, dnew_ref, m_ref):
    ea = ea_ref[...]
    acc = jnp.dot(ea, w1e_ref[...], preferred_element_type=F32)
    hmid = jnp.maximum(acc + ga_ref[...] + gb_ref[...] + b1_ref[...], 0.0)
    msg = jnp.maximum(
        jnp.dot(hmid, w2_ref[...], preferred_element_type=F32) + b2_ref[...], 0.0)
    logit = jnp.sum(msg * swr_ref[...], axis=1, keepdims=True) + sb_ref[...]
    gate = 1.0 / (1.0 + jnp.exp(-logit))
    dnew_ref[...] = ea + msg
    m_ref[...] = msg * gate


def _edge_mlp(ga, gb, ea, w1e, b1, w2, b2, swr, sb, tb):
    e, h = ea.shape
    grid = e // tb
    full = lambda i: (0, 0)
    tile = lambda i: (i, 0)
    return pl.pallas_call(
        _edge_body,
        grid=(grid,),
        in_specs=[
            pl.BlockSpec((tb, h), tile),
            pl.BlockSpec((tb, h), tile),
            pl.BlockSpec((tb, h), tile),
            pl.BlockSpec((h, h), full),
            pl.BlockSpec((1, h), full),
            pl.BlockSpec((h, h), full),
            pl.BlockSpec((1, h), full),
            pl.BlockSpec((1, h), full),
            pl.BlockSpec((1, 1), full),
        ],
        out_specs=[
            pl.BlockSpec((tb, h), tile),
            pl.BlockSpec((tb, h), tile),
        ],
        out_shape=[
            jax.ShapeDtypeStruct((e, h), F32),
            jax.ShapeDtypeStruct((e, h), F32),
        ],
    )(ga, gb, ea, w1e, b1, w2, b2, swr, sb)


def _make_scatter(n_pad, h, e, chunk):
    ew = e // _NW
    n_chunks = ew // chunk
    rows_per_tile = n_pad // _NS
    mesh = plsc.VectorSubcoreMesh(core_axis_name="c", subcore_axis_name="s")

    @functools.partial(
        pl.kernel,
        out_type=jax.ShapeDtypeStruct((_NC, n_pad, h), F32),
        mesh=mesh,
        scratch_types=[
            pltpu.VMEM_SHARED((n_pad, h), F32),
            pltpu.VMEM((chunk,), jnp.int32),
            pltpu.VMEM((chunk, h), F32),
        ],
    )
    def scatter_k(m_hbm, dst_hbm, part_hbm, acc_sh, idx_v, rows_v):
        cid = lax.axis_index("c")
        sid = lax.axis_index("s")
        wid = sid * _NC + cid
        base = wid * ew
        my_row0 = sid * rows_per_tile

        pieces = []
        r = 0
        while r < rows_per_tile:
            pieces.append((r, min(chunk, rows_per_tile - r)))
            r += chunk

        def zbody(i, carry):
            for j in range(h // 16):
                rows_v[i, pl.ds(j * 16, 16)] = jnp.zeros((16,), F32)
            return carry

        lax.fori_loop(0, chunk, zbody, 0)
        for r0, sz in pieces:
            pltpu.sync_copy(rows_v.at[pl.ds(0, sz)],
                            acc_sh.at[pl.ds(my_row0 + r0, sz)])
        plsc.subcore_barrier()

        def body(i, carry):
            off = base + i * chunk
            pltpu.sync_copy(dst_hbm.at[pl.ds(off, chunk)], idx_v)
            pltpu.sync_copy(m_hbm.at[pl.ds(off, chunk)], rows_v)
            pltpu.sync_copy(rows_v, acc_sh.at[idx_v], add=True)
            return carry

        lax.fori_loop(0, n_chunks, body, 0)
        plsc.subcore_barrier()

        for r0, sz in pieces:
            pltpu.sync_copy(acc_sh.at[pl.ds(my_row0 + r0, sz)],
                            rows_v.at[pl.ds(0, sz)])
            pltpu.sync_copy(rows_v.at[pl.ds(0, sz)],
                            part_hbm.at[cid, pl.ds(my_row0 + r0, sz)])

    return scatter_k


def _update_body(p0_ref, p1_ref, x_ref, u1_ref, ub1_ref, u2_ref, ub2_ref,
                 out_ref):
    x = x_ref[...]
    inp = p0_ref[...] + p1_ref[...] + x
    u = jnp.maximum(
        jnp.dot(inp, u1_ref[...], preferred_element_type=F32) + ub1_ref[...], 0.0)
    out_ref[...] = (
        jnp.dot(u, u2_ref[...], preferred_element_type=F32) + ub2_ref[...] + x)


def _node_update(p0, p1, x, u1, ub1, u2, ub2, bn):
    n, h = x.shape
    grid = n // bn
    full = lambda i: (0, 0)
    tile = lambda i: (i, 0)
    return pl.pallas_call(
        _update_body,
        grid=(grid,),
        in_specs=[
            pl.BlockSpec((bn, h), tile),
            pl.BlockSpec((bn, h), tile),
            pl.BlockSpec((bn, h), tile),
            pl.BlockSpec((h, h), full),
            pl.BlockSpec((1, h), full),
            pl.BlockSpec((h, h), full),
            pl.BlockSpec((1, h), full),
        ],
        out_specs=pl.BlockSpec((bn, h), tile),
        out_shape=jax.ShapeDtypeStruct((n, h), F32),
    )(p0, p1, x, u1, ub1, u2, ub2)


def kernel(x, edge_index, edge_attr, msg_W1, msg_b1, msg_W2, msg_b2,
           soft_W, soft_b, upd_W1, upd_b1, upd_W2, upd_b2):
    n, h = x.shape
    e = edge_index.shape[1]

    src = edge_index[0]
    dst = edge_index[1]
    w1a = msg_W1[:h]
    w1b = msg_W1[h:2 * h]
    w1e = msg_W1[2 * h:]
    b1 = msg_b1.reshape(1, h)
    b2 = msg_b2.reshape(1, h)
    swr = soft_W.reshape(1, h)
    sb = soft_b.reshape(1, 1)
    ub1 = upd_b1.reshape(1, h)
    ub2 = upd_b2.reshape(1, h)

    pa, pb = _node_proj(x, w1a, w1b, bn=1000)
    ga, gb = _make_gather(n, h, e, chunk=400)(pa, pb, src, dst)
    d_new, m = _edge_mlp(ga, gb, edge_attr, w1e, b1, msg_W2, b2, swr, sb,
                         tb=512)
    n_pad = _NS * ((n // _NS // 8 + 1) * 8)
    partials = _make_scatter(n_pad, h, e, chunk=200)(m, dst)
    out_feat = _node_update(partials[0], partials[1], x, upd_W1, ub1,
                            upd_W2, ub2, bn=1000)
    return (out_feat, d_new)


# speedup vs baseline: 2.9947x; 1.1931x over previous
import functools

import jax
import jax.numpy as jnp
from jax import lax
from jax.experimental import pallas as pl
from jax.experimental.pallas import tpu as pltpu
from jax.experimental.pallas import tpu_sc as plsc

F32 = jnp.float32

_NC = 2
_NS = 16
_NW = _NC * _NS


def _proj_body(x_ref, wa_ref, wb_ref, pa_ref, pb_ref):
    x = x_ref[...]
    pa_ref[...] = jnp.dot(x, wa_ref[...], preferred_element_type=F32)
    pb_ref[...] = jnp.dot(x, wb_ref[...], preferred_element_type=F32)


def _node_proj(x, w1a, w1b, bn):
    n, h = x.shape
    grid = n // bn
    return pl.pallas_call(
        _proj_body,
        grid=(grid,),
        in_specs=[
            pl.BlockSpec((bn, h), lambda i: (i, 0)),
            pl.BlockSpec((h, h), lambda i: (0, 0)),
            pl.BlockSpec((h, h), lambda i: (0, 0)),
        ],
        out_specs=[
            pl.BlockSpec((bn, h), lambda i: (i, 0)),
            pl.BlockSpec((bn, h), lambda i: (i, 0)),
        ],
        out_shape=[
            jax.ShapeDtypeStruct((n, h), F32),
            jax.ShapeDtypeStruct((n, h), F32),
        ],
    )(x, w1a, w1b)


def _make_gather(n, h, e, chunk):
    ew = e // _NW
    n_chunks = ew // chunk
    mesh = plsc.VectorSubcoreMesh(core_axis_name="c", subcore_axis_name="s")

    @functools.partial(
        pl.kernel,
        out_type=(
            jax.ShapeDtypeStruct((e, h), F32),
            jax.ShapeDtypeStruct((e, h), F32),
        ),
        mesh=mesh,
        scratch_types=[
            pltpu.VMEM((chunk,), jnp.int32),
            pltpu.VMEM((chunk,), jnp.int32),
            pltpu.VMEM((chunk, h), F32),
            pltpu.VMEM((chunk, h), F32),
            pltpu.SemaphoreType.DMA,
            pltpu.SemaphoreType.DMA,
        ],
    )
    def gather_k(pa_hbm, pb_hbm, src_hbm, dst_hbm, ga_hbm, gb_hbm,
                 idx_a, idx_b, rows_a, rows_b, sem_a, sem_b):
        wid = lax.axis_index("s") * _NC + lax.axis_index("c")
        base = wid * ew

        def body(i, carry):
            off = base + i * chunk
            pltpu.sync_copy(src_hbm.at[pl.ds(off, chunk)], idx_a)
            pltpu.sync_copy(dst_hbm.at[pl.ds(off, chunk)], idx_b)
            cp_a = pltpu.async_copy(pa_hbm.at[idx_a], rows_a, sem_a)
            cp_b = pltpu.async_copy(pb_hbm.at[idx_b], rows_b, sem_b)
            cp_a.wait()
            cp_b.wait()
            pltpu.sync_copy(rows_a, ga_hbm.at[pl.ds(off, chunk)])
            pltpu.sync_copy(rows_b, gb_hbm.at[pl.ds(off, chunk)])
            return carry

        lax.fori_loop(0, n_chunks, body, 0)

    return gather_k


def _edge_body(ga_ref, gb_ref, ea_ref, w1e_ref, b1_ref, w2_ref, b2_ref,
               swr_ref, sb_ref, dnew_ref, m_ref):
    ea = ea_ref[...]
    acc = jnp.dot(ea, w1e_ref[...], preferred_element_type=F32)
    hmid = jnp.maximum(acc + ga_ref[...] + gb_ref[...] + b1_ref[...], 0.0)
    msg = jnp.maximum(
        jnp.dot(hmid, w2_ref[...], preferred_element_type=F32) + b2_ref[...], 0.0)
    logit = jnp.sum(msg * swr_ref[...], axis=1, keepdims=True) + sb_ref[...]
    gate = 1.0 / (1.0 + jnp.exp(-logit))
    dnew_ref[...] = ea + msg
    m_ref[...] = msg * gate


def _edge_body_chain(ga_ref, gb_ref, ea_ref, dprev_ref, w1e_ref, b1_ref,
                     w2_ref, b2_ref, swr_ref, sb_ref, dnew_ref, m_ref):
    del dprev_ref
    _edge_body(ga_ref, gb_ref, ea_ref, w1e_ref, b1_ref, w2_ref, b2_ref,
               swr_ref, sb_ref, dnew_ref, m_ref)


def _edge_mlp_slice(ga, gb, ea, dprev, w1e, b1, w2, b2, swr, sb, tb,
                    slice_idx):
    e_full, h = ea.shape
    es = ga.shape[0]
    grid = es // tb
    off = slice_idx * grid
    full = lambda i: (0, 0)
    tile = lambda i: (i, 0)
    shifted = lambda i: (i + off, 0)
    weight_specs = [
        pl.BlockSpec((h, h), full),
        pl.BlockSpec((1, h), full),
        pl.BlockSpec((h, h), full),
        pl.BlockSpec((1, h), full),
        pl.BlockSpec((1, h), full),
        pl.BlockSpec((1, 1), full),
    ]
    out_specs = [
        pl.BlockSpec((tb, h), shifted),
        pl.BlockSpec((tb, h), tile),
    ]
    out_shape = [
        jax.ShapeDtypeStruct((e_full, h), F32),
        jax.ShapeDtypeStruct((es, h), F32),
    ]
    slice_specs = [
        pl.BlockSpec((tb, h), tile),
        pl.BlockSpec((tb, h), tile),
        pl.BlockSpec((tb, h), shifted),
    ]
    if dprev is None:
        return pl.pallas_call(
            _edge_body,
            grid=(grid,),
            in_specs=slice_specs + weight_specs,
            out_specs=out_specs,
            out_shape=out_shape,
        )(ga, gb, ea, w1e, b1, w2, b2, swr, sb)
    return pl.pallas_call(
        _edge_body_chain,
        grid=(grid,),
        in_specs=slice_specs + [pl.BlockSpec((tb, h), shifted)] + weight_specs,
        out_specs=out_specs,
        out_shape=out_shape,
        input_output_aliases={3: 0},
    )(ga, gb, ea, dprev, w1e, b1, w2, b2, swr, sb)


def _make_scatter(n_pad, h, e_slice, n_slices, chunk):
    ew = e_slice // _NW
    n_chunks = ew // chunk
    rows_per_tile = n_pad // _NS
    mesh = plsc.VectorSubcoreMesh(core_axis_name="c", subcore_axis_name="s")

    @functools.partial(
        pl.kernel,
        out_type=jax.ShapeDtypeStruct((_NC, n_pad, h), F32),
        mesh=mesh,
        scratch_types=[
            pltpu.VMEM_SHARED((n_pad, h), F32),
            pltpu.VMEM((chunk,), jnp.int32),
            pltpu.VMEM((chunk, h), F32),
        ],
    )
    def scatter_k(*refs):
        m_refs = refs[:n_slices]
        dst_refs = refs[n_slices:2 * n_slices]
        part_hbm = refs[2 * n_slices]
        acc_sh, idx_v, rows_v = refs[2 * n_slices + 1:]
        cid = lax.axis_index("c")
        sid = lax.axis_index("s")
        wid = sid * _NC + cid
        base = wid * ew
        my_row0 = sid * rows_per_tile

        pieces = []
        r = 0
        while r < rows_per_tile:
            pieces.append((r, min(chunk, rows_per_tile - r)))
            r += chunk

        def zbody(i, carry):
            for j in range(h // 16):
                rows_v[i, pl.ds(j * 16, 16)] = jnp.zeros((16,), F32)
            return carry

        lax.fori_loop(0, chunk, zbody, 0)
        for r0, sz in pieces:
            pltpu.sync_copy(rows_v.at[pl.ds(0, sz)],
                            acc_sh.at[pl.ds(my_row0 + r0, sz)])
        plsc.subcore_barrier()

        for s in range(n_slices):
            m_hbm = m_refs[s]
            dst_hbm = dst_refs[s]

            def body(i, carry, m_hbm=m_hbm, dst_hbm=dst_hbm):
                off = base + i * chunk
                pltpu.sync_copy(dst_hbm.at[pl.ds(off, chunk)], idx_v)
                pltpu.sync_copy(m_hbm.at[pl.ds(off, chunk)], rows_v)
                pltpu.sync_copy(rows_v, acc_sh.at[idx_v], add=True)
                return carry

            lax.fori_loop(0, n_chunks, body, 0)
        plsc.subcore_barrier()

        for r0, sz in pieces:
            pltpu.sync_copy(acc_sh.at[pl.ds(my_row0 + r0, sz)],
                            rows_v.at[pl.ds(0, sz)])
            pltpu.sync_copy(rows_v.at[pl.ds(0, sz)],
                            part_hbm.at[cid, pl.ds(my_row0 + r0, sz)])

    return scatter_k


def _make_update_body(n_parts):
    def _update_body(*refs):
        parts = refs[:n_parts]
        x_ref, u1_ref, ub1_ref, u2_ref, ub2_ref, out_ref = refs[n_parts:]
        x = x_ref[...]
        inp = x
        for p in parts:
            inp = inp + p[...]
        u = jnp.maximum(
            jnp.dot(inp, u1_ref[...], preferred_element_type=F32)
            + ub1_ref[...], 0.0)
        out_ref[...] = (
            jnp.dot(u, u2_ref[...], preferred_element_type=F32)
            + ub2_ref[...] + x)
    return _update_body


def _node_update(parts, x, u1, ub1, u2, ub2, bn):
    n, h = x.shape
    grid = n // bn
    full = lambda i: (0, 0)
    tile = lambda i: (i, 0)
    return pl.pallas_call(
        _make_update_body(len(parts)),
        grid=(grid,),
        in_specs=[pl.BlockSpec((bn, h), tile) for _ in parts] + [
            pl.BlockSpec((bn, h), tile),
            pl.BlockSpec((h, h), full),
            pl.BlockSpec((1, h), full),
            pl.BlockSpec((h, h), full),
            pl.BlockSpec((1, h), full),
        ],
        out_specs=pl.BlockSpec((bn, h), tile),
        out_shape=jax.ShapeDtypeStruct((n, h), F32),
    )(*parts, x, u1, ub1, u2, ub2)


def kernel(x, edge_index, edge_attr, msg_W1, msg_b1, msg_W2, msg_b2,
           soft_W, soft_b, upd_W1, upd_b1, upd_W2, upd_b2):
    n, h = x.shape
    e = edge_index.shape[1]

    src = edge_index[0]
    dst = edge_index[1]
    w1a = msg_W1[:h]
    w1b = msg_W1[h:2 * h]
    w1e = msg_W1[2 * h:]
    b1 = msg_b1.reshape(1, h)
    b2 = msg_b2.reshape(1, h)
    swr = soft_W.reshape(1, h)
    sb = soft_b.reshape(1, 1)
    ub1 = upd_b1.reshape(1, h)
    ub2 = upd_b2.reshape(1, h)

    pa, pb = _node_proj(x, w1a, w1b, bn=1000)

    n_slices = 5
    es = e // n_slices
    gather_fn = _make_gather(n, h, es, chunk=400)
    srcs = [src[i * es:(i + 1) * es] for i in range(n_slices)]
    dsts = [dst[i * es:(i + 1) * es] for i in range(n_slices)]
    gs = [gather_fn(pa, pb, srcs[i], dsts[i]) for i in range(n_slices)]

    d_new = None
    ms = []
    for i in range(n_slices):
        ga, gb = gs[i]
        d_new, m_i = _edge_mlp_slice(ga, gb, edge_attr, d_new, w1e, b1,
                                     msg_W2, b2, swr, sb, tb=512,
                                     slice_idx=i)
        ms.append(m_i)

    n_pad = _NS * ((n // _NS // 8 + 1) * 8)
    ga_slices, gb_slices = 3, 2
    part_a = _make_scatter(n_pad, h, es, ga_slices, chunk=200)(
        *ms[:ga_slices], *dsts[:ga_slices])
    part_b = _make_scatter(n_pad, h, es, gb_slices, chunk=200)(
        *ms[ga_slices:], *dsts[ga_slices:])
    parts = [part_a[0], part_a[1], part_b[0], part_b[1]]
    out_feat = _node_update(parts, x, upd_W1, ub1, upd_W2, ub2, bn=1000)
    return (out_feat, d_new)


# speedup vs baseline: 3.3715x; 1.1258x over previous
import functools

import jax
import jax.numpy as jnp
from jax import lax
from jax.experimental import pallas as pl
from jax.experimental.pallas import tpu as pltpu
from jax.experimental.pallas import tpu_sc as plsc

F32 = jnp.float32

_NC = 2
_NS = 16
_NW = _NC * _NS


def _proj_body(x_ref, wa_ref, wb_ref, pa_ref, pb_ref):
    x = x_ref[...]
    pa_ref[...] = jnp.dot(x, wa_ref[...], preferred_element_type=F32)
    pb_ref[...] = jnp.dot(x, wb_ref[...], preferred_element_type=F32)


def _node_proj(x, w1a, w1b, bn):
    n, h = x.shape
    grid = n // bn
    return pl.pallas_call(
        _proj_body,
        grid=(grid,),
        in_specs=[
            pl.BlockSpec((bn, h), lambda i: (i, 0)),
            pl.BlockSpec((h, h), lambda i: (0, 0)),
            pl.BlockSpec((h, h), lambda i: (0, 0)),
        ],
        out_specs=[
            pl.BlockSpec((bn, h), lambda i: (i, 0)),
            pl.BlockSpec((bn, h), lambda i: (i, 0)),
        ],
        out_shape=[
            jax.ShapeDtypeStruct((n, h), F32),
            jax.ShapeDtypeStruct((n, h), F32),
        ],
    )(x, w1a, w1b)


def _make_gather(n, h, e_slice, chunk):
    ew = e_slice // _NW
    n_chunks = ew // chunk
    mesh = plsc.VectorSubcoreMesh(core_axis_name="c", subcore_axis_name="s")

    @functools.partial(
        pl.kernel,
        out_type=jax.ShapeDtypeStruct((e_slice, h), F32),
        mesh=mesh,
        scratch_types=[
            pltpu.VMEM((ew,), jnp.int32),
            pltpu.VMEM((ew,), jnp.int32),
            pltpu.VMEM((chunk, h), F32),
            pltpu.VMEM((chunk, h), F32),
            pltpu.VMEM((chunk, h), F32),
            pltpu.VMEM((chunk, h), F32),
            pltpu.SemaphoreType.DMA,
            pltpu.SemaphoreType.DMA,
            pltpu.SemaphoreType.DMA,
            pltpu.SemaphoreType.DMA,
            pltpu.SemaphoreType.DMA,
            pltpu.SemaphoreType.DMA,
        ],
    )
    def gather_k(pa_hbm, pb_hbm, src_hbm, dst_hbm, g_hbm,
                 idx_a, idx_b, ra0, ra1, rb0, rb1,
                 sa0, sa1, sb0, sb1, sw0, sw1):
        wid = lax.axis_index("s") * _NC + lax.axis_index("c")
        base = wid * ew
        pltpu.sync_copy(src_hbm.at[pl.ds(base, ew)], idx_a)
        pltpu.sync_copy(dst_hbm.at[pl.ds(base, ew)], idx_b)
        ra = (ra0, ra1)
        rb = (rb0, rb1)
        sa = (sa0, sa1)
        sb = (sb0, sb1)
        sw = (sw0, sw1)
        cps = {}
        for i in range(n_chunks + 1):
            j = i % 2
            if i >= 2:
                cps[("w", i - 2)].wait()
            if i < n_chunks:
                ia = idx_a.at[pl.ds(i * chunk, chunk)]
                ib = idx_b.at[pl.ds(i * chunk, chunk)]
                cps[("a", i)] = pltpu.async_copy(pa_hbm.at[ia], ra[j], sa[j])
                cps[("b", i)] = pltpu.async_copy(pb_hbm.at[ib], rb[j], sb[j])
            if i >= 1:
                k = (i - 1) % 2
                cps[("a", i - 1)].wait()
                cps[("b", i - 1)].wait()
                a_buf, b_buf = ra[k], rb[k]

                def addbody(r, carry, a_buf=a_buf, b_buf=b_buf):
                    for c in range(h // 16):
                        sl = pl.ds(c * 16, 16)
                        a_buf[r, sl] = a_buf[r, sl] + b_buf[r, sl]
                    return carry

                lax.fori_loop(0, chunk, addbody, 0)
                cps[("w", i - 1)] = pltpu.async_copy(
                    a_buf, g_hbm.at[pl.ds(base + (i - 1) * chunk, chunk)],
                    sw[k])
        cps[("w", n_chunks - 1)].wait()

    return gather_k


def _edge_body(g_ref, ea_ref, w1e_ref, b1_ref, w2_ref, b2_ref,
               swr_ref, sb_ref, dnew_ref, m_ref):
    ea = ea_ref[...]
    acc = jnp.dot(ea, w1e_ref[...], preferred_element_type=F32)
    hmid = jnp.maximum(acc + g_ref[...] + b1_ref[...], 0.0)
    msg = jnp.maximum(
        jnp.dot(hmid, w2_ref[...], preferred_element_type=F32) + b2_ref[...], 0.0)
    logit = jnp.sum(msg * swr_ref[...], axis=1, keepdims=True) + sb_ref[...]
    gate = 1.0 / (1.0 + jnp.exp(-logit))
    dnew_ref[...] = ea + msg
    m_ref[...] = msg * gate


def _edge_body_chain(g_ref, ea_ref, dprev_ref, w1e_ref, b1_ref,
                     w2_ref, b2_ref, swr_ref, sb_ref, dnew_ref, m_ref):
    del dprev_ref
    _edge_body(g_ref, ea_ref, w1e_ref, b1_ref, w2_ref, b2_ref,
               swr_ref, sb_ref, dnew_ref, m_ref)


def _edge_mlp_slice(g, ea, dprev, w1e, b1, w2, b2, swr, sb, tb,
                    slice_idx):
    e_full, h = ea.shape
    es = g.shape[0]
    grid = es // tb
    off = slice_idx * grid
    full = lambda i: (0, 0)
    tile = lambda i: (i, 0)
    shifted = lambda i: (i + off, 0)
    weight_specs = [
        pl.BlockSpec((h, h), full),
        pl.BlockSpec((1, h), full),
        pl.BlockSpec((h, h), full),
        pl.BlockSpec((1, h), full),
        pl.BlockSpec((1, h), full),
        pl.BlockSpec((1, 1), full),
    ]
    out_specs = [
        pl.BlockSpec((tb, h), shifted),
        pl.BlockSpec((tb, h), tile),
    ]
    out_shape = [
        jax.ShapeDtypeStruct((e_full, h), F32),
        jax.ShapeDtypeStruct((es, h), F32),
    ]
    slice_specs = [
        pl.BlockSpec((tb, h), tile),
        pl.BlockSpec((tb, h), shifted),
    ]
    if dprev is None:
        return pl.pallas_call(
            _edge_body,
            grid=(grid,),
            in_specs=slice_specs + weight_specs,
            out_specs=out_specs,
            out_shape=out_shape,
        )(g, ea, w1e, b1, w2, b2, swr, sb)
    return pl.pallas_call(
        _edge_body_chain,
        grid=(grid,),
        in_specs=slice_specs + [pl.BlockSpec((tb, h), shifted)] + weight_specs,
        out_specs=out_specs,
        out_shape=out_shape,
        input_output_aliases={2: 0},
    )(g, ea, dprev, w1e, b1, w2, b2, swr, sb)


def _make_scatter(n_pad, h, e_slice, n_slices, chunk):
    ew = e_slice // _NW
    n_chunks = ew // chunk
    rows_per_tile = n_pad // _NS
    mesh = plsc.VectorSubcoreMesh(core_axis_name="c", subcore_axis_name="s")

    n_tasks = n_slices * n_chunks

    @functools.partial(
        pl.kernel,
        out_type=jax.ShapeDtypeStruct((_NC, n_pad, h), F32),
        mesh=mesh,
        scratch_types=[
            pltpu.VMEM_SHARED((n_pad, h), F32),
            pltpu.VMEM((n_tasks, chunk), jnp.int32),
            pltpu.VMEM((chunk, h), F32),
            pltpu.VMEM((chunk, h), F32),
            pltpu.SemaphoreType.DMA,
            pltpu.SemaphoreType.DMA,
            pltpu.SemaphoreType.DMA,
            pltpu.SemaphoreType.DMA,
            pltpu.SemaphoreType.DMA,
            pltpu.SemaphoreType.DMA,
        ],
    )
    def scatter_k(*refs):
        m_refs = refs[:n_slices]
        dst_refs = refs[n_slices:2 * n_slices]
        part_hbm = refs[2 * n_slices]
        acc_sh, idx2d, rb0, rb1, si0, si1, sl0, sl1, ss0, ss1 = (
            refs[2 * n_slices + 1:])
        rbuf = (rb0, rb1)
        si = (si0, si1)
        sl = (sl0, sl1)
        ss = (ss0, ss1)
        cid = lax.axis_index("c")
        sid = lax.axis_index("s")
        wid = sid * _NC + cid
        base = wid * ew
        my_row0 = sid * rows_per_tile

        pieces = []
        r = 0
        while r < rows_per_tile:
            pieces.append((r, min(chunk, rows_per_tile - r)))
            r += chunk

        def zbody(i, carry):
            for j in range(h // 16):
                rb0[i, pl.ds(j * 16, 16)] = jnp.zeros((16,), F32)
            return carry

        lax.fori_loop(0, chunk, zbody, 0)
        for r0, sz in pieces:
            pltpu.sync_copy(rb0.at[pl.ds(0, sz)],
                            acc_sh.at[pl.ds(my_row0 + r0, sz)])
        plsc.subcore_barrier()

        tasks = [(s, c) for s in range(n_slices) for c in range(n_chunks)]
        cps = {}
        for t in range(n_tasks + 1):
            j = t % 2
            if t >= 2:
                cps[("s", t - 2)].wait()
            if t < n_tasks:
                s, c = tasks[t]
                off = base + c * chunk
                cps[("i", t)] = pltpu.async_copy(
                    dst_refs[s].at[pl.ds(off, chunk)], idx2d.at[t], si[j])
                cps[("m", t)] = pltpu.async_copy(
                    m_refs[s].at[pl.ds(off, chunk)], rbuf[j], sl[j])
            if t >= 1:
                k = (t - 1) % 2
                cps[("i", t - 1)].wait()
                cps[("m", t - 1)].wait()
                cps[("s", t - 1)] = pltpu.async_copy(
                    rbuf[k], acc_sh.at[idx2d.at[t - 1]], ss[k], add=True)
        cps[("s", n_tasks - 1)].wait()
        plsc.subcore_barrier()

        wcps = {}
        np_ = len(pieces)
        for t in range(np_ + 1):
            j = t % 2
            if t >= 2:
                wcps[("w", t - 2)].wait()
            if t < np_:
                r0, sz = pieces[t]
                wcps[("r", t)] = pltpu.async_copy(
                    acc_sh.at[pl.ds(my_row0 + r0, sz)],
                    rbuf[j].at[pl.ds(0, sz)], sl[j])
            if t >= 1:
                k = (t - 1) % 2
                r0, sz = pieces[t - 1]
                wcps[("r", t - 1)].wait()
                wcps[("w", t - 1)] = pltpu.async_copy(
                    rbuf[k].at[pl.ds(0, sz)],
                    part_hbm.at[cid, pl.ds(my_row0 + r0, sz)], ss[k])
        wcps[("w", np_ - 1)].wait()

    return scatter_k


def _make_update_body(n_parts):
    def _update_body(*refs):
        parts = refs[:n_parts]
        x_ref, u1_ref, ub1_ref, u2_ref, ub2_ref, out_ref = refs[n_parts:]
        x = x_ref[...]
        inp = x
        for p in parts:
            inp = inp + p[...]
        u = jnp.maximum(
            jnp.dot(inp, u1_ref[...], preferred_element_type=F32)
            + ub1_ref[...], 0.0)
        out_ref[...] = (
            jnp.dot(u, u2_ref[...], preferred_element_type=F32)
            + ub2_ref[...] + x)
    return _update_body


def _node_update(parts, x, u1, ub1, u2, ub2, bn):
    n, h = x.shape
    grid = n // bn
    full = lambda i: (0, 0)
    tile = lambda i: (i, 0)
    return pl.pallas_call(
        _make_update_body(len(parts)),
        grid=(grid,),
        in_specs=[pl.BlockSpec((bn, h), tile) for _ in parts] + [
            pl.BlockSpec((bn, h), tile),
            pl.BlockSpec((h, h), full),
            pl.BlockSpec((1, h), full),
            pl.BlockSpec((h, h), full),
            pl.BlockSpec((1, h), full),
        ],
        out_specs=pl.BlockSpec((bn, h), tile),
        out_shape=jax.ShapeDtypeStruct((n, h), F32),
    )(*parts, x, u1, ub1, u2, ub2)


def kernel(x, edge_index, edge_attr, msg_W1, msg_b1, msg_W2, msg_b2,
           soft_W, soft_b, upd_W1, upd_b1, upd_W2, upd_b2):
    n, h = x.shape
    e = edge_index.shape[1]

    src = edge_index[0]
    dst = edge_index[1]
    w1a = msg_W1[:h]
    w1b = msg_W1[h:2 * h]
    w1e = msg_W1[2 * h:]
    b1 = msg_b1.reshape(1, h)
    b2 = msg_b2.reshape(1, h)
    swr = soft_W.reshape(1, h)
    sb = soft_b.reshape(1, 1)
    ub1 = upd_b1.reshape(1, h)
    ub2 = upd_b2.reshape(1, h)

    pa, pb = _node_proj(x, w1a, w1b, bn=1000)

    n_slices = 5
    es = e // n_slices
    gather_fn = _make_gather(n, h, es, chunk=200)
    srcs = [src[i * es:(i + 1) * es] for i in range(n_slices)]
    dsts = [dst[i * es:(i + 1) * es] for i in range(n_slices)]
    gs = [gather_fn(pa, pb, srcs[i], dsts[i]) for i in range(n_slices)]

    d_new = None
    ms = []
    for i in range(n_slices):
        d_new, m_i = _edge_mlp_slice(gs[i], edge_attr, d_new, w1e, b1,
                                     msg_W2, b2, swr, sb, tb=512,
                                     slice_idx=i)
        ms.append(m_i)

    n_pad = _NS * ((n // _NS // 8 + 1) * 8)
    ga_slices, gb_slices = 3, 2
    part_a = _make_scatter(n_pad, h, es, ga_slices, chunk=80)(
        *ms[:ga_slices], *dsts[:ga_slices])
    part_b = _make_scatter(n_pad, h, es, gb_slices, chunk=80)(
        *ms[ga_slices:], *dsts[ga_slices:])
    parts = [part_a[0], part_a[1], part_b[0], part_b[1]]
    out_feat = _node_update(parts, x, upd_W1, ub1, upd_W2, ub2, bn=1000)
    return (out_feat, d_new)


# speedup vs baseline: 3.4797x; 1.0321x over previous
import functools

import jax
import jax.numpy as jnp
from jax import lax
from jax.experimental import pallas as pl
from jax.experimental.pallas import tpu as pltpu
from jax.experimental.pallas import tpu_sc as plsc

F32 = jnp.float32

_NC = 2
_NS = 16
_NW = _NC * _NS


def _proj_body(x_ref, wa_ref, wb_ref, pa_ref, pb_ref):
    x = x_ref[...]
    pa_ref[...] = jnp.dot(x, wa_ref[...], preferred_element_type=F32)
    pb_ref[...] = jnp.dot(x, wb_ref[...], preferred_element_type=F32)


def _node_proj(x, w1a, w1b, bn):
    n, h = x.shape
    grid = n // bn
    return pl.pallas_call(
        _proj_body,
        grid=(grid,),
        in_specs=[
            pl.BlockSpec((bn, h), lambda i: (i, 0)),
            pl.BlockSpec((h, h), lambda i: (0, 0)),
            pl.BlockSpec((h, h), lambda i: (0, 0)),
        ],
        out_specs=[
            pl.BlockSpec((bn, h), lambda i: (i, 0)),
            pl.BlockSpec((bn, h), lambda i: (i, 0)),
        ],
        out_shape=[
            jax.ShapeDtypeStruct((n, h), F32),
            jax.ShapeDtypeStruct((n, h), F32),
        ],
    )(x, w1a, w1b)


def _make_gather(n, h, e_slice, chunk):
    ew = e_slice // _NW
    n_chunks = ew // chunk
    mesh = plsc.VectorSubcoreMesh(core_axis_name="c", subcore_axis_name="s")

    @functools.partial(
        pl.kernel,
        out_type=jax.ShapeDtypeStruct((e_slice, h), F32),
        mesh=mesh,
        scratch_types=[
            pltpu.VMEM((ew,), jnp.int32),
            pltpu.VMEM((ew,), jnp.int32),
            pltpu.VMEM((chunk, h), F32),
            pltpu.VMEM((chunk, h), F32),
            pltpu.VMEM((chunk, h), F32),
            pltpu.VMEM((chunk, h), F32),
            pltpu.SemaphoreType.DMA,
            pltpu.SemaphoreType.DMA,
            pltpu.SemaphoreType.DMA,
            pltpu.SemaphoreType.DMA,
            pltpu.SemaphoreType.DMA,
            pltpu.SemaphoreType.DMA,
        ],
    )
    def gather_k(pa_hbm, pb_hbm, src_hbm, dst_hbm, g_hbm,
                 idx_a, idx_b, ra0, ra1, rb0, rb1,
                 sa0, sa1, sb0, sb1, sw0, sw1):
        wid = lax.axis_index("s") * _NC + lax.axis_index("c")
        base = wid * ew
        pltpu.sync_copy(src_hbm.at[pl.ds(base, ew)], idx_a)
        pltpu.sync_copy(dst_hbm.at[pl.ds(base, ew)], idx_b)
        ra = (ra0, ra1)
        rb = (rb0, rb1)
        sa = (sa0, sa1)
        sb = (sb0, sb1)
        sw = (sw0, sw1)
        cps = {}
        for i in range(n_chunks + 1):
            j = i % 2
            if i >= 2:
                cps[("w", i - 2)].wait()
            if i < n_chunks:
                ia = idx_a.at[pl.ds(i * chunk, chunk)]
                ib = idx_b.at[pl.ds(i * chunk, chunk)]
                cps[("a", i)] = pltpu.async_copy(pa_hbm.at[ia], ra[j], sa[j])
                cps[("b", i)] = pltpu.async_copy(pb_hbm.at[ib], rb[j], sb[j])
            if i >= 1:
                k = (i - 1) % 2
                cps[("a", i - 1)].wait()
                cps[("b", i - 1)].wait()
                a_buf, b_buf = ra[k], rb[k]

                def addbody(r, carry, a_buf=a_buf, b_buf=b_buf):
                    for c in range(h // 16):
                        sl = pl.ds(c * 16, 16)
                        a_buf[r, sl] = a_buf[r, sl] + b_buf[r, sl]
                    return carry

                lax.fori_loop(0, chunk, addbody, 0)
                cps[("w", i - 1)] = pltpu.async_copy(
                    a_buf, g_hbm.at[pl.ds(base + (i - 1) * chunk, chunk)],
                    sw[k])
        cps[("w", n_chunks - 1)].wait()

    return gather_k


def _make_edge_body(tb, h, row_off, grid_n, has_prev):
    def body(*refs):
        if has_prev:
            (g_ref, ea_ref, dprev_ref, w1e_ref, b1_ref, w2_ref, b2_ref,
             swr_ref, sb_ref, dnew_any, m_ref, db0, db1, ds0, ds1) = refs
            del dprev_ref
        else:
            (g_ref, ea_ref, w1e_ref, b1_ref, w2_ref, b2_ref,
             swr_ref, sb_ref, dnew_any, m_ref, db0, db1, ds0, ds1) = refs
        step = pl.program_id(0)
        ea = ea_ref[...]
        acc = jnp.dot(ea, w1e_ref[...], preferred_element_type=F32)
        hmid = jnp.maximum(acc + g_ref[...] + b1_ref[...], 0.0)
        msg = jnp.maximum(
            jnp.dot(hmid, w2_ref[...], preferred_element_type=F32)
            + b2_ref[...], 0.0)
        logit = jnp.sum(msg * swr_ref[...], axis=1, keepdims=True) + sb_ref[...]
        gate = 1.0 / (1.0 + jnp.exp(-logit))
        m_ref[...] = msg * gate
        dn = ea + msg

        rows = pl.ds((row_off + step) * tb, tb)
        parity = lax.rem(step, 2)
        for p, (buf, sem) in enumerate(((db0, ds0), (db1, ds1))):
            @pl.when((parity == p) & (step >= 2))
            def _(buf=buf, sem=sem):
                pltpu.make_async_copy(buf, dnew_any.at[rows], sem).wait()

            @pl.when(parity == p)
            def _(buf=buf, sem=sem):
                buf[...] = dn
                pltpu.make_async_copy(buf, dnew_any.at[rows], sem).start()

        @pl.when(step == grid_n - 1)
        def _():
            pltpu.make_async_copy(db0, dnew_any.at[rows], ds0).wait()
            pltpu.make_async_copy(db1, dnew_any.at[rows], ds1).wait()

    return body


def _edge_mlp_slice(g, ea, dprev, w1e, b1, w2, b2, swr, sb, tb,
                    slice_idx):
    e_full, h = ea.shape
    es = g.shape[0]
    grid = es // tb
    off = slice_idx * grid
    full = lambda i: (0, 0)
    tile = lambda i: (i, 0)
    shifted = lambda i: (i + off, 0)
    weight_specs = [
        pl.BlockSpec((h, h), full),
        pl.BlockSpec((1, h), full),
        pl.BlockSpec((h, h), full),
        pl.BlockSpec((1, h), full),
        pl.BlockSpec((1, h), full),
        pl.BlockSpec((1, 1), full),
    ]
    any_spec = pl.BlockSpec(memory_space=pl.ANY)
    out_specs = [
        any_spec,
        pl.BlockSpec((tb, h), tile),
    ]
    out_shape = [
        jax.ShapeDtypeStruct((e_full, h), F32),
        jax.ShapeDtypeStruct((es, h), F32),
    ]
    slice_specs = [
        pl.BlockSpec((tb, h), tile),
        pl.BlockSpec((tb, h), shifted),
    ]
    scratch = [
        pltpu.VMEM((tb, h), F32),
        pltpu.VMEM((tb, h), F32),
        pltpu.SemaphoreType.DMA,
        pltpu.SemaphoreType.DMA,
    ]
    if dprev is None:
        return pl.pallas_call(
            _make_edge_body(tb, h, off, grid, has_prev=False),
            grid=(grid,),
            in_specs=slice_specs + weight_specs,
            out_specs=out_specs,
            out_shape=out_shape,
            scratch_shapes=scratch,
        )(g, ea, w1e, b1, w2, b2, swr, sb)
    return pl.pallas_call(
        _make_edge_body(tb, h, off, grid, has_prev=True),
        grid=(grid,),
        in_specs=slice_specs + [any_spec] + weight_specs,
        out_specs=out_specs,
        out_shape=out_shape,
        scratch_shapes=scratch,
        input_output_aliases={2: 0},
    )(g, ea, dprev, w1e, b1, w2, b2, swr, sb)


def _make_scatter(n_pad, h, e_slice, n_slices, chunk):
    ew = e_slice // _NW
    n_chunks = ew // chunk
    rows_per_tile = n_pad // _NS
    mesh = plsc.VectorSubcoreMesh(core_axis_name="c", subcore_axis_name="s")

    n_tasks = n_slices * n_chunks

    @functools.partial(
        pl.kernel,
        out_type=jax.ShapeDtypeStruct((_NC, n_pad, h), F32),
        mesh=mesh,
        scratch_types=[
            pltpu.VMEM_SHARED((n_pad, h), F32),
            pltpu.VMEM((n_tasks, chunk), jnp.int32),
            pltpu.VMEM((chunk, h), F32),
            pltpu.VMEM((chunk, h), F32),
            pltpu.SemaphoreType.DMA,
            pltpu.SemaphoreType.DMA,
            pltpu.SemaphoreType.DMA,
            pltpu.SemaphoreType.DMA,
            pltpu.SemaphoreType.DMA,
            pltpu.SemaphoreType.DMA,
        ],
    )
    def scatter_k(*refs):
        m_refs = refs[:n_slices]
        dst_refs = refs[n_slices:2 * n_slices]
        part_hbm = refs[2 * n_slices]
        acc_sh, idx2d, rb0, rb1, si0, si1, sl0, sl1, ss0, ss1 = (
            refs[2 * n_slices + 1:])
        rbuf = (rb0, rb1)
        si = (si0, si1)
        sl = (sl0, sl1)
        ss = (ss0, ss1)
        cid = lax.axis_index("c")
        sid = lax.axis_index("s")
        wid = sid * _NC + cid
        base = wid * ew
        my_row0 = sid * rows_per_tile

        pieces = []
        r = 0
        while r < rows_per_tile:
            pieces.append((r, min(chunk, rows_per_tile - r)))
            r += chunk

        def zbody(i, carry):
            for j in range(h // 16):
                rb0[i, pl.ds(j * 16, 16)] = jnp.zeros((16,), F32)
            return carry

        lax.fori_loop(0, chunk, zbody, 0)
        for r0, sz in pieces:
            pltpu.sync_copy(rb0.at[pl.ds(0, sz)],
                            acc_sh.at[pl.ds(my_row0 + r0, sz)])
        plsc.subcore_barrier()

        tasks = [(s, c) for s in range(n_slices) for c in range(n_chunks)]
        cps = {}
        for t in range(n_tasks + 1):
            j = t % 2
            if t >= 2:
                cps[("s", t - 2)].wait()
            if t < n_tasks:
                s, c = tasks[t]
                off = base + c * chunk
                cps[("i", t)] = pltpu.async_copy(
                    dst_refs[s].at[pl.ds(off, chunk)], idx2d.at[t], si[j])
                cps[("m", t)] = pltpu.async_copy(
                    m_refs[s].at[pl.ds(off, chunk)], rbuf[j], sl[j])
            if t >= 1:
                k = (t - 1) % 2
                cps[("i", t - 1)].wait()
                cps[("m", t - 1)].wait()
                cps[("s", t - 1)] = pltpu.async_copy(
                    rbuf[k], acc_sh.at[idx2d.at[t - 1]], ss[k], add=True)
        cps[("s", n_tasks - 1)].wait()
        plsc.subcore_barrier()

        wcps = {}
        np_ = len(pieces)
        for t in range(np_ + 1):
            j = t % 2
            if t >= 2:
                wcps[("w", t - 2)].wait()
            if t < np_:
                r0, sz = pieces[t]
                wcps[("r", t)] = pltpu.async_copy(
                    acc_sh.at[pl.ds(my_row0 + r0, sz)],
                    rbuf[j].at[pl.ds(0, sz)], sl[j])
            if t >= 1:
                k = (t - 1) % 2
                r0, sz = pieces[t - 1]
                wcps[("r", t - 1)].wait()
                wcps[("w", t - 1)] = pltpu.async_copy(
                    rbuf[k].at[pl.ds(0, sz)],
                    part_hbm.at[cid, pl.ds(my_row0 + r0, sz)], ss[k])
        wcps[("w", np_ - 1)].wait()

    return scatter_k


def _make_update_body(n_parts):
    def _update_body(*refs):
        parts = refs[:n_parts]
        x_ref, u1_ref, ub1_ref, u2_ref, ub2_ref, out_ref = refs[n_parts:]
        x = x_ref[...]
        inp = x
        for p in parts:
            inp = inp + p[...]
        u = jnp.maximum(
            jnp.dot(inp, u1_ref[...], preferred_element_type=F32)
            + ub1_ref[...], 0.0)
        out_ref[...] = (
            jnp.dot(u, u2_ref[...], preferred_element_type=F32)
            + ub2_ref[...] + x)
    return _update_body


def _node_update(parts, x, u1, ub1, u2, ub2, bn):
    n, h = x.shape
    grid = n // bn
    full = lambda i: (0, 0)
    tile = lambda i: (i, 0)
    return pl.pallas_call(
        _make_update_body(len(parts)),
        grid=(grid,),
        in_specs=[pl.BlockSpec((bn, h), tile) for _ in parts] + [
            pl.BlockSpec((bn, h), tile),
            pl.BlockSpec((h, h), full),
            pl.BlockSpec((1, h), full),
            pl.BlockSpec((h, h), full),
            pl.BlockSpec((1, h), full),
        ],
        out_specs=pl.BlockSpec((bn, h), tile),
        out_shape=jax.ShapeDtypeStruct((n, h), F32),
    )(*parts, x, u1, ub1, u2, ub2)


def kernel(x, edge_index, edge_attr, msg_W1, msg_b1, msg_W2, msg_b2,
           soft_W, soft_b, upd_W1, upd_b1, upd_W2, upd_b2):
    n, h = x.shape
    e = edge_index.shape[1]

    src = edge_index[0]
    dst = edge_index[1]
    w1a = msg_W1[:h]
    w1b = msg_W1[h:2 * h]
    w1e = msg_W1[2 * h:]
    b1 = msg_b1.reshape(1, h)
    b2 = msg_b2.reshape(1, h)
    swr = soft_W.reshape(1, h)
    sb = soft_b.reshape(1, 1)
    ub1 = upd_b1.reshape(1, h)
    ub2 = upd_b2.reshape(1, h)

    pa, pb = _node_proj(x, w1a, w1b, bn=1000)

    n_slices = 5
    es = e // n_slices
    gather_fn = _make_gather(n, h, es, chunk=200)
    srcs = [src[i * es:(i + 1) * es] for i in range(n_slices)]
    dsts = [dst[i * es:(i + 1) * es] for i in range(n_slices)]
    gs = [gather_fn(pa, pb, srcs[i], dsts[i]) for i in range(n_slices)]

    d_new = None
    ms = []
    for i in range(n_slices):
        d_new, m_i = _edge_mlp_slice(gs[i], edge_attr, d_new, w1e, b1,
                                     msg_W2, b2, swr, sb, tb=512,
                                     slice_idx=i)
        ms.append(m_i)

    n_pad = _NS * ((n // _NS // 8 + 1) * 8)
    ga_slices, gb_slices = 3, 2
    part_a = _make_scatter(n_pad, h, es, ga_slices, chunk=80)(
        *ms[:ga_slices], *dsts[:ga_slices])
    part_b = _make_scatter(n_pad, h, es, gb_slices, chunk=80)(
        *ms[ga_slices:], *dsts[ga_slices:])
    parts = [part_a[0], part_a[1], part_b[0], part_b[1]]
    out_feat = _node_update(parts, x, upd_W1, ub1, upd_W2, ub2, bn=1000)
    return (out_feat, d_new)


# speedup vs baseline: 3.6777x; 1.0569x over previous
import functools

import jax
import jax.numpy as jnp
from jax import lax
from jax.experimental import pallas as pl
from jax.experimental.pallas import tpu as pltpu
from jax.experimental.pallas import tpu_sc as plsc

F32 = jnp.float32

_NC = 2
_NS = 16
_NW = _NC * _NS


def _proj_body(x_ref, wa_ref, wb_ref, pa_ref, pb_ref):
    x = x_ref[...]
    pa_ref[...] = jnp.dot(x, wa_ref[...], preferred_element_type=F32)
    pb_ref[...] = jnp.dot(x, wb_ref[...], preferred_element_type=F32)


def _node_proj(x, w1a, w1b, bn):
    n, h = x.shape
    grid = n // bn
    return pl.pallas_call(
        _proj_body,
        grid=(grid,),
        in_specs=[
            pl.BlockSpec((bn, h), lambda i: (i, 0)),
            pl.BlockSpec((h, h), lambda i: (0, 0)),
            pl.BlockSpec((h, h), lambda i: (0, 0)),
        ],
        out_specs=[
            pl.BlockSpec((bn, h), lambda i: (i, 0)),
            pl.BlockSpec((bn, h), lambda i: (i, 0)),
        ],
        out_shape=[
            jax.ShapeDtypeStruct((n, h), F32),
            jax.ShapeDtypeStruct((n, h), F32),
        ],
    )(x, w1a, w1b)


def _make_gather(n, h, e_slice, chunk):
    ew = e_slice // _NW
    n_chunks = ew // chunk
    mesh = plsc.VectorSubcoreMesh(core_axis_name="c", subcore_axis_name="s")

    @functools.partial(
        pl.kernel,
        out_type=jax.ShapeDtypeStruct((e_slice, h), F32),
        mesh=mesh,
        scratch_types=[
            pltpu.VMEM((ew,), jnp.int32),
            pltpu.VMEM((ew,), jnp.int32),
            pltpu.VMEM((chunk, h), F32),
            pltpu.VMEM((chunk, h), F32),
            pltpu.VMEM((chunk, h), F32),
            pltpu.VMEM((chunk, h), F32),
            pltpu.SemaphoreType.DMA,
            pltpu.SemaphoreType.DMA,
            pltpu.SemaphoreType.DMA,
            pltpu.SemaphoreType.DMA,
            pltpu.SemaphoreType.DMA,
            pltpu.SemaphoreType.DMA,
        ],
    )
    def gather_k(pa_hbm, pb_hbm, src_hbm, dst_hbm, g_hbm,
                 idx_a, idx_b, ra0, ra1, rb0, rb1,
                 sa0, sa1, sb0, sb1, sw0, sw1):
        wid = lax.axis_index("s") * _NC + lax.axis_index("c")
        base = wid * ew
        pltpu.sync_copy(src_hbm.at[pl.ds(base, ew)], idx_a)
        pltpu.sync_copy(dst_hbm.at[pl.ds(base, ew)], idx_b)
        ra = (ra0, ra1)
        rb = (rb0, rb1)
        sa = (sa0, sa1)
        sb = (sb0, sb1)
        sw = (sw0, sw1)
        cps = {}
        for i in range(n_chunks + 1):
            j = i % 2
            if i >= 2:
                cps[("w", i - 2)].wait()
            if i < n_chunks:
                ia = idx_a.at[pl.ds(i * chunk, chunk)]
                ib = idx_b.at[pl.ds(i * chunk, chunk)]
                cps[("a", i)] = pltpu.async_copy(pa_hbm.at[ia], ra[j], sa[j])
                cps[("b", i)] = pltpu.async_copy(pb_hbm.at[ib], rb[j], sb[j])
            if i >= 1:
                k = (i - 1) % 2
                cps[("a", i - 1)].wait()
                cps[("b", i - 1)].wait()
                a_buf, b_buf = ra[k], rb[k]

                def addbody(r, carry, a_buf=a_buf, b_buf=b_buf):
                    for c in range(h // 16):
                        sl = pl.ds(c * 16, 16)
                        a_buf[r, sl] = a_buf[r, sl] + b_buf[r, sl]
                    return carry

                lax.fori_loop(0, chunk, addbody, 0)
                cps[("w", i - 1)] = pltpu.async_copy(
                    a_buf, g_hbm.at[pl.ds(base + (i - 1) * chunk, chunk)],
                    sw[k])
        cps[("w", n_chunks - 1)].wait()

    return gather_k


def _make_edge_body(tb, h, row_off, grid_n, has_prev):
    def body(*refs):
        if has_prev:
            (g_ref, ea_ref, dprev_ref, w1e_ref, b1_ref, w2_ref, b2_ref,
             swr_ref, sb_ref, dnew_any, m_ref, db0, db1, ds0, ds1) = refs
            del dprev_ref
        else:
            (g_ref, ea_ref, w1e_ref, b1_ref, w2_ref, b2_ref,
             swr_ref, sb_ref, dnew_any, m_ref, db0, db1, ds0, ds1) = refs
        step = pl.program_id(0)
        ea = ea_ref[...]
        acc = jnp.dot(ea, w1e_ref[...], preferred_element_type=F32)
        hmid = jnp.maximum(acc + g_ref[...] + b1_ref[...], 0.0)
        msg = jnp.maximum(
            jnp.dot(hmid, w2_ref[...], preferred_element_type=F32)
            + b2_ref[...], 0.0)
        logit = jnp.sum(msg * swr_ref[...], axis=1, keepdims=True) + sb_ref[...]
        gate = 1.0 / (1.0 + jnp.exp(-logit))
        m_ref[...] = msg * gate
        dn = ea + msg

        rows = pl.ds((row_off + step) * tb, tb)
        parity = lax.rem(step, 2)
        for p, (buf, sem) in enumerate(((db0, ds0), (db1, ds1))):
            @pl.when((parity == p) & (step >= 2))
            def _(buf=buf, sem=sem):
                pltpu.make_async_copy(buf, dnew_any.at[rows], sem).wait()

            @pl.when(parity == p)
            def _(buf=buf, sem=sem):
                buf[...] = dn
                pltpu.make_async_copy(buf, dnew_any.at[rows], sem).start()

        @pl.when(step == grid_n - 1)
        def _():
            pltpu.make_async_copy(db0, dnew_any.at[rows], ds0).wait()
            pltpu.make_async_copy(db1, dnew_any.at[rows], ds1).wait()

    return body


def _edge_mlp_slice(g, ea, dprev, w1e, b1, w2, b2, swr, sb, tb,
                    slice_idx):
    e_full, h = ea.shape
    es = g.shape[0]
    grid = es // tb
    off = slice_idx * grid
    full = lambda i: (0, 0)
    tile = lambda i: (i, 0)
    shifted = lambda i: (i + off, 0)
    weight_specs = [
        pl.BlockSpec((h, h), full),
        pl.BlockSpec((1, h), full),
        pl.BlockSpec((h, h), full),
        pl.BlockSpec((1, h), full),
        pl.BlockSpec((1, h), full),
        pl.BlockSpec((1, 1), full),
    ]
    any_spec = pl.BlockSpec(memory_space=pl.ANY)
    out_specs = [
        any_spec,
        pl.BlockSpec((tb, h), tile),
    ]
    out_shape = [
        jax.ShapeDtypeStruct((e_full, h), F32),
        jax.ShapeDtypeStruct((es, h), F32),
    ]
    slice_specs = [
        pl.BlockSpec((tb, h), tile),
        pl.BlockSpec((tb, h), shifted),
    ]
    scratch = [
        pltpu.VMEM((tb, h), F32),
        pltpu.VMEM((tb, h), F32),
        pltpu.SemaphoreType.DMA,
        pltpu.SemaphoreType.DMA,
    ]
    if dprev is None:
        return pl.pallas_call(
            _make_edge_body(tb, h, off, grid, has_prev=False),
            grid=(grid,),
            in_specs=slice_specs + weight_specs,
            out_specs=out_specs,
            out_shape=out_shape,
            scratch_shapes=scratch,
        )(g, ea, w1e, b1, w2, b2, swr, sb)
    return pl.pallas_call(
        _make_edge_body(tb, h, off, grid, has_prev=True),
        grid=(grid,),
        in_specs=slice_specs + [any_spec] + weight_specs,
        out_specs=out_specs,
        out_shape=out_shape,
        scratch_shapes=scratch,
        input_output_aliases={2: 0},
    )(g, ea, dprev, w1e, b1, w2, b2, swr, sb)


def _make_scatter(n_pad, h, e_slice, n_slices, chunk):
    ew = e_slice // _NW
    n_chunks = ew // chunk
    rows_per_tile = n_pad // _NS
    mesh = plsc.VectorSubcoreMesh(core_axis_name="c", subcore_axis_name="s")

    n_tasks = n_slices * n_chunks

    @functools.partial(
        pl.kernel,
        out_type=jax.ShapeDtypeStruct((_NC, n_pad, h), F32),
        mesh=mesh,
        scratch_types=[
            pltpu.VMEM_SHARED((n_pad, h), F32),
            pltpu.VMEM((n_tasks, chunk), jnp.int32),
            pltpu.VMEM((chunk, h), F32),
            pltpu.VMEM((chunk, h), F32),
            pltpu.SemaphoreType.DMA,
            pltpu.SemaphoreType.DMA,
            pltpu.SemaphoreType.DMA,
            pltpu.SemaphoreType.DMA,
            pltpu.SemaphoreType.DMA,
            pltpu.SemaphoreType.DMA,
        ],
    )
    def scatter_k(*refs):
        m_refs = refs[:n_slices]
        dst_refs = refs[n_slices:2 * n_slices]
        part_hbm = refs[2 * n_slices]
        acc_sh, idx2d, rb0, rb1, si0, si1, sl0, sl1, ss0, ss1 = (
            refs[2 * n_slices + 1:])
        rbuf = (rb0, rb1)
        si = (si0, si1)
        sl = (sl0, sl1)
        ss = (ss0, ss1)
        cid = lax.axis_index("c")
        sid = lax.axis_index("s")
        wid = sid * _NC + cid
        base = wid * ew
        my_row0 = sid * rows_per_tile

        pieces = []
        r = 0
        while r < rows_per_tile:
            pieces.append((r, min(chunk, rows_per_tile - r)))
            r += chunk

        def zbody(i, carry):
            for j in range(h // 16):
                rb0[i, pl.ds(j * 16, 16)] = jnp.zeros((16,), F32)
            return carry

        lax.fori_loop(0, chunk, zbody, 0)
        for r0, sz in pieces:
            pltpu.sync_copy(rb0.at[pl.ds(0, sz)],
                            acc_sh.at[pl.ds(my_row0 + r0, sz)])
        plsc.subcore_barrier()

        tasks = [(s, c) for s in range(n_slices) for c in range(n_chunks)]
        cps = {}
        for t in range(n_tasks + 1):
            j = t % 2
            if t >= 2:
                cps[("s", t - 2)].wait()
            if t < n_tasks:
                s, c = tasks[t]
                off = base + c * chunk
                cps[("i", t)] = pltpu.async_copy(
                    dst_refs[s].at[pl.ds(off, chunk)], idx2d.at[t], si[j])
                cps[("m", t)] = pltpu.async_copy(
                    m_refs[s].at[pl.ds(off, chunk)], rbuf[j], sl[j])
            if t >= 1:
                k = (t - 1) % 2
                cps[("i", t - 1)].wait()
                cps[("m", t - 1)].wait()
                cps[("s", t - 1)] = pltpu.async_copy(
                    rbuf[k], acc_sh.at[idx2d.at[t - 1]], ss[k], add=True)
        cps[("s", n_tasks - 1)].wait()
        plsc.subcore_barrier()

        wcps = {}
        np_ = len(pieces)
        for t in range(np_ + 1):
            j = t % 2
            if t >= 2:
                wcps[("w", t - 2)].wait()
            if t < np_:
                r0, sz = pieces[t]
                wcps[("r", t)] = pltpu.async_copy(
                    acc_sh.at[pl.ds(my_row0 + r0, sz)],
                    rbuf[j].at[pl.ds(0, sz)], sl[j])
            if t >= 1:
                k = (t - 1) % 2
                r0, sz = pieces[t - 1]
                wcps[("r", t - 1)].wait()
                wcps[("w", t - 1)] = pltpu.async_copy(
                    rbuf[k].at[pl.ds(0, sz)],
                    part_hbm.at[cid, pl.ds(my_row0 + r0, sz)], ss[k])
        wcps[("w", np_ - 1)].wait()

    return scatter_k


def _make_update_body(n_parts):
    def _update_body(*refs):
        parts = refs[:n_parts]
        x_ref, u1_ref, ub1_ref, u2_ref, ub2_ref, out_ref = refs[n_parts:]
        x = x_ref[...]
        inp = x
        for p in parts:
            inp = inp + p[...]
        u = jnp.maximum(
            jnp.dot(inp, u1_ref[...], preferred_element_type=F32)
            + ub1_ref[...], 0.0)
        out_ref[...] = (
            jnp.dot(u, u2_ref[...], preferred_element_type=F32)
            + ub2_ref[...] + x)
    return _update_body


def _node_update(parts, x, u1, ub1, u2, ub2, bn):
    n, h = x.shape
    grid = n // bn
    full = lambda i: (0, 0)
    tile = lambda i: (i, 0)
    return pl.pallas_call(
        _make_update_body(len(parts)),
        grid=(grid,),
        in_specs=[pl.BlockSpec((bn, h), tile) for _ in parts] + [
            pl.BlockSpec((bn, h), tile),
            pl.BlockSpec((h, h), full),
            pl.BlockSpec((1, h), full),
            pl.BlockSpec((h, h), full),
            pl.BlockSpec((1, h), full),
        ],
        out_specs=pl.BlockSpec((bn, h), tile),
        out_shape=jax.ShapeDtypeStruct((n, h), F32),
    )(*parts, x, u1, ub1, u2, ub2)


def kernel(x, edge_index, edge_attr, msg_W1, msg_b1, msg_W2, msg_b2,
           soft_W, soft_b, upd_W1, upd_b1, upd_W2, upd_b2):
    n, h = x.shape
    e = edge_index.shape[1]

    src = edge_index[0]
    dst = edge_index[1]
    w1a = msg_W1[:h]
    w1b = msg_W1[h:2 * h]
    w1e = msg_W1[2 * h:]
    b1 = msg_b1.reshape(1, h)
    b2 = msg_b2.reshape(1, h)
    swr = soft_W.reshape(1, h)
    sb = soft_b.reshape(1, 1)
    ub1 = upd_b1.reshape(1, h)
    ub2 = upd_b2.reshape(1, h)

    pa, pb = _node_proj(x, w1a, w1b, bn=1000)

    n_slices = 5
    es = e // n_slices
    gather_fn = _make_gather(n, h, es, chunk=200)
    srcs = [src[i * es:(i + 1) * es] for i in range(n_slices)]
    dsts = [dst[i * es:(i + 1) * es] for i in range(n_slices)]
    gs = [gather_fn(pa, pb, srcs[i], dsts[i]) for i in range(n_slices)]

    d_new = None
    ms = []
    for i in range(n_slices):
        d_new, m_i = _edge_mlp_slice(gs[i], edge_attr, d_new, w1e, b1,
                                     msg_W2, b2, swr, sb, tb=1000,
                                     slice_idx=i)
        ms.append(m_i)

    n_pad = _NS * ((n // _NS // 8 + 1) * 8)
    ga_slices, gb_slices = 3, 2
    part_a = _make_scatter(n_pad, h, es, ga_slices, chunk=80)(
        *ms[:ga_slices], *dsts[:ga_slices])
    part_b = _make_scatter(n_pad, h, es, gb_slices, chunk=80)(
        *ms[ga_slices:], *dsts[ga_slices:])
    parts = [part_a[0], part_a[1], part_b[0], part_b[1]]
    out_feat = _node_update(parts, x, upd_W1, ub1, upd_W2, ub2, bn=1000)
    return (out_feat, d_new)


# speedup vs baseline: 4.1695x; 1.1337x over previous
import functools

import jax
import jax.numpy as jnp
from jax import lax
from jax.experimental import pallas as pl
from jax.experimental.pallas import tpu as pltpu
from jax.experimental.pallas import tpu_sc as plsc

F32 = jnp.float32
BF16 = jnp.bfloat16

_NC = 2
_NS = 16
_NW = _NC * _NS


def _proj_body(x_ref, wa_ref, wb_ref, pa_ref, pb_ref):
    x = x_ref[...]
    pa_ref[...] = jnp.dot(x, wa_ref[...], preferred_element_type=F32)
    pb_ref[...] = jnp.dot(x, wb_ref[...], preferred_element_type=F32)


def _node_proj(x, w1a, w1b, bn):
    n, h = x.shape
    grid = n // bn
    return pl.pallas_call(
        _proj_body,
        grid=(grid,),
        in_specs=[
            pl.BlockSpec((bn, h), lambda i: (i, 0)),
            pl.BlockSpec((h, h), lambda i: (0, 0)),
            pl.BlockSpec((h, h), lambda i: (0, 0)),
        ],
        out_specs=[
            pl.BlockSpec((bn, h), lambda i: (i, 0)),
            pl.BlockSpec((bn, h), lambda i: (i, 0)),
        ],
        out_shape=[
            jax.ShapeDtypeStruct((n, h), F32),
            jax.ShapeDtypeStruct((n, h), F32),
        ],
    )(x, w1a, w1b)


def _make_gather(n, h, e_slice, chunk):
    ew = e_slice // _NW
    n_chunks = ew // chunk
    mesh = plsc.VectorSubcoreMesh(core_axis_name="c", subcore_axis_name="s")

    @functools.partial(
        pl.kernel,
        out_type=jax.ShapeDtypeStruct((e_slice, h), F32),
        mesh=mesh,
        scratch_types=[
            pltpu.VMEM((ew,), jnp.int32),
            pltpu.VMEM((ew,), jnp.int32),
            pltpu.VMEM((chunk, h), F32),
            pltpu.VMEM((chunk, h), F32),
            pltpu.VMEM((chunk, h), F32),
            pltpu.VMEM((chunk, h), F32),
            pltpu.SemaphoreType.DMA,
            pltpu.SemaphoreType.DMA,
            pltpu.SemaphoreType.DMA,
            pltpu.SemaphoreType.DMA,
            pltpu.SemaphoreType.DMA,
            pltpu.SemaphoreType.DMA,
        ],
    )
    def gather_k(pa_hbm, pb_hbm, src_hbm, dst_hbm, g_hbm,
                 idx_a, idx_b, ra0, ra1, rb0, rb1,
                 sa0, sa1, sb0, sb1, sw0, sw1):
        wid = lax.axis_index("s") * _NC + lax.axis_index("c")
        base = wid * ew
        pltpu.sync_copy(src_hbm.at[pl.ds(base, ew)], idx_a)
        pltpu.sync_copy(dst_hbm.at[pl.ds(base, ew)], idx_b)
        ra = (ra0, ra1)
        rb = (rb0, rb1)
        sa = (sa0, sa1)
        sb = (sb0, sb1)
        sw = (sw0, sw1)
        cps = {}
        for i in range(n_chunks + 1):
            j = i % 2
            if i >= 2:
                cps[("w", i - 2)].wait()
            if i < n_chunks:
                ia = idx_a.at[pl.ds(i * chunk, chunk)]
                ib = idx_b.at[pl.ds(i * chunk, chunk)]
                cps[("a", i)] = pltpu.async_copy(pa_hbm.at[ia], ra[j], sa[j])
                cps[("b", i)] = pltpu.async_copy(pb_hbm.at[ib], rb[j], sb[j])
            if i >= 1:
                k = (i - 1) % 2
                cps[("a", i - 1)].wait()
                cps[("b", i - 1)].wait()
                a_buf, b_buf = ra[k], rb[k]

                def addbody(r, carry, a_buf=a_buf, b_buf=b_buf):
                    for c in range(h // 16):
                        sl = pl.ds(c * 16, 16)
                        a_buf[r, sl] = a_buf[r, sl] + b_buf[r, sl]
                    return carry

                lax.fori_loop(0, chunk, addbody, 0)
                cps[("w", i - 1)] = pltpu.async_copy(
                    a_buf, g_hbm.at[pl.ds(base + (i - 1) * chunk, chunk)],
                    sw[k])
        cps[("w", n_chunks - 1)].wait()

    return gather_k


def _make_edge_body(tb, h, row_off, grid_n, has_prev):
    def body(*refs):
        if has_prev:
            (g_ref, ea_ref, dprev_ref, w1e_ref, b1_ref, w2_ref,
             b2_ref, swr_ref, sb_ref, dnew_any, m_ref, db0, db1, ds0,
             ds1) = refs
            del dprev_ref
        else:
            (g_ref, ea_ref, w1e_ref, b1_ref, w2_ref, b2_ref,
             swr_ref, sb_ref, dnew_any, m_ref, db0, db1, ds0, ds1) = refs
        step = pl.program_id(0)
        ea = ea_ref[...]
        acc = jnp.dot(ea.astype(BF16), w1e_ref[...],
                      preferred_element_type=F32)
        hmid = jnp.maximum(acc + g_ref[...] + b1_ref[...], 0.0)
        msg = jnp.maximum(
            jnp.dot(hmid.astype(BF16), w2_ref[...],
                    preferred_element_type=F32)
            + b2_ref[...], 0.0)
        logit = jnp.sum(msg * swr_ref[...], axis=1, keepdims=True) + sb_ref[...]
        gate = 1.0 / (1.0 + jnp.exp(-logit))
        m_ref[...] = msg * gate
        dn = ea + msg

        rows = pl.ds((row_off + step) * tb, tb)
        parity = lax.rem(step, 2)
        for p, (buf, sem) in enumerate(((db0, ds0), (db1, ds1))):
            @pl.when((parity == p) & (step >= 2))
            def _(buf=buf, sem=sem):
                pltpu.make_async_copy(buf, dnew_any.at[rows], sem).wait()

            @pl.when(parity == p)
            def _(buf=buf, sem=sem):
                buf[...] = dn
                pltpu.make_async_copy(buf, dnew_any.at[rows], sem).start()

        @pl.when(step == grid_n - 1)
        def _():
            pltpu.make_async_copy(db0, dnew_any.at[rows], ds0).wait()
            pltpu.make_async_copy(db1, dnew_any.at[rows], ds1).wait()

    return body


def _edge_mlp_slice(g, ea, dprev, w1e, b1, w2, b2, swr, sb, tb,
                    slice_idx):
    e_full, h = ea.shape
    es = g.shape[0]
    grid = es // tb
    off = slice_idx * grid
    full = lambda i: (0, 0)
    tile = lambda i: (i, 0)
    shifted = lambda i: (i + off, 0)
    weight_specs = [
        pl.BlockSpec((h, h), full),
        pl.BlockSpec((1, h), full),
        pl.BlockSpec((h, h), full),
        pl.BlockSpec((1, h), full),
        pl.BlockSpec((1, h), full),
        pl.BlockSpec((1, 1), full),
    ]
    any_spec = pl.BlockSpec(memory_space=pl.ANY)
    out_specs = [
        any_spec,
        pl.BlockSpec((tb, h), tile),
    ]
    out_shape = [
        jax.ShapeDtypeStruct((e_full, h), F32),
        jax.ShapeDtypeStruct((es, h), F32),
    ]
    slice_specs = [
        pl.BlockSpec((tb, h), tile),
        pl.BlockSpec((tb, h), shifted),
    ]
    scratch = [
        pltpu.VMEM((tb, h), F32),
        pltpu.VMEM((tb, h), F32),
        pltpu.SemaphoreType.DMA,
        pltpu.SemaphoreType.DMA,
    ]
    if dprev is None:
        return pl.pallas_call(
            _make_edge_body(tb, h, off, grid, has_prev=False),
            grid=(grid,),
            in_specs=slice_specs + weight_specs,
            out_specs=out_specs,
            out_shape=out_shape,
            scratch_shapes=scratch,
        )(g, ea, w1e, b1, w2, b2, swr, sb)
    return pl.pallas_call(
        _make_edge_body(tb, h, off, grid, has_prev=True),
        grid=(grid,),
        in_specs=slice_specs + [any_spec] + weight_specs,
        out_specs=out_specs,
        out_shape=out_shape,
        scratch_shapes=scratch,
        input_output_aliases={2: 0},
    )(g, ea, dprev, w1e, b1, w2, b2, swr, sb)


def _make_scatter(n_pad, h, e_slice, n_slices, chunk):
    ew = e_slice // _NW
    n_chunks = ew // chunk
    rows_per_tile = n_pad // _NS
    mesh = plsc.VectorSubcoreMesh(core_axis_name="c", subcore_axis_name="s")

    n_tasks = n_slices * n_chunks

    @functools.partial(
        pl.kernel,
        out_type=jax.ShapeDtypeStruct((_NC, n_pad, h), F32),
        mesh=mesh,
        scratch_types=[
            pltpu.VMEM_SHARED((n_pad, h), F32),
            pltpu.VMEM((n_tasks, chunk), jnp.int32),
            pltpu.VMEM((chunk, h), F32),
            pltpu.VMEM((chunk, h), F32),
            pltpu.SemaphoreType.DMA,
            pltpu.SemaphoreType.DMA,
            pltpu.SemaphoreType.DMA,
            pltpu.SemaphoreType.DMA,
            pltpu.SemaphoreType.DMA,
            pltpu.SemaphoreType.DMA,
        ],
    )
    def scatter_k(*refs):
        m_refs = refs[:n_slices]
        dst_refs = refs[n_slices:2 * n_slices]
        part_hbm = refs[2 * n_slices]
        acc_sh, idx2d, rb0, rb1, si0, si1, sl0, sl1, ss0, ss1 = (
            refs[2 * n_slices + 1:])
        rbuf = (rb0, rb1)
        si = (si0, si1)
        sl = (sl0, sl1)
        ss = (ss0, ss1)
        cid = lax.axis_index("c")
        sid = lax.axis_index("s")
        wid = sid * _NC + cid
        base = wid * ew
        my_row0 = sid * rows_per_tile

        pieces = []
        r = 0
        while r < rows_per_tile:
            pieces.append((r, min(chunk, rows_per_tile - r)))
            r += chunk

        def zbody(i, carry):
            for j in range(h // 16):
                rb0[i, pl.ds(j * 16, 16)] = jnp.zeros((16,), F32)
            return carry

        lax.fori_loop(0, chunk, zbody, 0)
        for r0, sz in pieces:
            pltpu.sync_copy(rb0.at[pl.ds(0, sz)],
                            acc_sh.at[pl.ds(my_row0 + r0, sz)])
        plsc.subcore_barrier()

        tasks = [(s, c) for s in range(n_slices) for c in range(n_chunks)]
        cps = {}
        for t in range(n_tasks + 1):
            j = t % 2
            if t >= 2:
                cps[("s", t - 2)].wait()
            if t < n_tasks:
                s, c = tasks[t]
                off = base + c * chunk
                cps[("i", t)] = pltpu.async_copy(
                    dst_refs[s].at[pl.ds(off, chunk)], idx2d.at[t], si[j])
                cps[("m", t)] = pltpu.async_copy(
                    m_refs[s].at[pl.ds(off, chunk)], rbuf[j], sl[j])
            if t >= 1:
                k = (t - 1) % 2
                cps[("i", t - 1)].wait()
                cps[("m", t - 1)].wait()
                cps[("s", t - 1)] = pltpu.async_copy(
                    rbuf[k], acc_sh.at[idx2d.at[t - 1]], ss[k], add=True)
        cps[("s", n_tasks - 1)].wait()
        plsc.subcore_barrier()

        wcps = {}
        np_ = len(pieces)
        for t in range(np_ + 1):
            j = t % 2
            if t >= 2:
                wcps[("w", t - 2)].wait()
            if t < np_:
                r0, sz = pieces[t]
                wcps[("r", t)] = pltpu.async_copy(
                    acc_sh.at[pl.ds(my_row0 + r0, sz)],
                    rbuf[j].at[pl.ds(0, sz)], sl[j])
            if t >= 1:
                k = (t - 1) % 2
                r0, sz = pieces[t - 1]
                wcps[("r", t - 1)].wait()
                wcps[("w", t - 1)] = pltpu.async_copy(
                    rbuf[k].at[pl.ds(0, sz)],
                    part_hbm.at[cid, pl.ds(my_row0 + r0, sz)], ss[k])
        wcps[("w", np_ - 1)].wait()

    return scatter_k


def _make_update_body(n_parts):
    def _update_body(*refs):
        parts = refs[:n_parts]
        x_ref, u1_ref, ub1_ref, u2_ref, ub2_ref, out_ref = refs[n_parts:]
        x = x_ref[...]
        inp = x
        for p in parts:
            inp = inp + p[...].astype(F32)
        u = jnp.maximum(
            jnp.dot(inp, u1_ref[...], preferred_element_type=F32)
            + ub1_ref[...], 0.0)
        out_ref[...] = (
            jnp.dot(u, u2_ref[...], preferred_element_type=F32)
            + ub2_ref[...] + x)
    return _update_body


def _node_update(parts, x, u1, ub1, u2, ub2, bn):
    n, h = x.shape
    grid = n // bn
    full = lambda i: (0, 0)
    tile = lambda i: (i, 0)
    return pl.pallas_call(
        _make_update_body(len(parts)),
        grid=(grid,),
        in_specs=[pl.BlockSpec((bn, h), tile) for _ in parts] + [
            pl.BlockSpec((bn, h), tile),
            pl.BlockSpec((h, h), full),
            pl.BlockSpec((1, h), full),
            pl.BlockSpec((h, h), full),
            pl.BlockSpec((1, h), full),
        ],
        out_specs=pl.BlockSpec((bn, h), tile),
        out_shape=jax.ShapeDtypeStruct((n, h), F32),
    )(*parts, x, u1, ub1, u2, ub2)


def kernel(x, edge_index, edge_attr, msg_W1, msg_b1, msg_W2, msg_b2,
           soft_W, soft_b, upd_W1, upd_b1, upd_W2, upd_b2):
    n, h = x.shape
    e = edge_index.shape[1]

    src = edge_index[0]
    dst = edge_index[1]
    w1a = msg_W1[:h]
    w1b = msg_W1[h:2 * h]
    w1e = msg_W1[2 * h:]
    b1 = msg_b1.reshape(1, h)
    b2 = msg_b2.reshape(1, h)
    swr = soft_W.reshape(1, h)
    sb = soft_b.reshape(1, 1)
    ub1 = upd_b1.reshape(1, h)
    ub2 = upd_b2.reshape(1, h)

    pa, pb = _node_proj(x, w1a, w1b, bn=2000)

    n_slices = 5
    es = e // n_slices
    gather_fn = _make_gather(n, h, es, chunk=200)
    srcs = [src[i * es:(i + 1) * es] for i in range(n_slices)]
    dsts = [dst[i * es:(i + 1) * es] for i in range(n_slices)]
    gs = [gather_fn(pa, pb, srcs[i], dsts[i]) for i in range(n_slices)]

    w1e_bf = w1e.astype(BF16)
    w2_bf = msg_W2.astype(BF16)
    d_new = None
    ms = []
    for i in range(n_slices):
        d_new, m_i = _edge_mlp_slice(gs[i], edge_attr, d_new, w1e_bf,
                                     b1, w2_bf, b2, swr, sb, tb=800,
                                     slice_idx=i)
        ms.append(m_i)

    n_pad = _NS * ((n // _NS // 8 + 1) * 8)
    ga_slices, gb_slices = 3, 2
    part_a = _make_scatter(n_pad, h, es, ga_slices, chunk=80)(
        *ms[:ga_slices], *dsts[:ga_slices])
    part_b = _make_scatter(n_pad, h, es, gb_slices, chunk=80)(
        *ms[ga_slices:], *dsts[ga_slices:])
    parts = [part_a[0], part_a[1], part_b[0], part_b[1]]
    out_feat = _node_update(parts, x, upd_W1, ub1, upd_W2, ub2, bn=2000)
    return (out_feat, d_new)


# speedup vs baseline: 4.2973x; 1.0307x over previous
import functools

import jax
import jax.numpy as jnp
from jax import lax
from jax.experimental import pallas as pl
from jax.experimental.pallas import tpu as pltpu
from jax.experimental.pallas import tpu_sc as plsc

F32 = jnp.float32
BF16 = jnp.bfloat16

_NC = 2
_NS = 16
_NW = _NC * _NS


def _proj_body(x_ref, wa_ref, wb_ref, pa_ref, pb_ref):
    x = x_ref[...]
    pa_ref[...] = jnp.dot(x, wa_ref[...], preferred_element_type=F32)
    pb_ref[...] = jnp.dot(x, wb_ref[...], preferred_element_type=F32)


def _node_proj(x, w1a, w1b, bn):
    n, h = x.shape
    grid = n // bn
    return pl.pallas_call(
        _proj_body,
        grid=(grid,),
        in_specs=[
            pl.BlockSpec((bn, h), lambda i: (i, 0)),
            pl.BlockSpec((h, h), lambda i: (0, 0)),
            pl.BlockSpec((h, h), lambda i: (0, 0)),
        ],
        out_specs=[
            pl.BlockSpec((bn, h), lambda i: (i, 0)),
            pl.BlockSpec((bn, h), lambda i: (i, 0)),
        ],
        out_shape=[
            jax.ShapeDtypeStruct((n, h), F32),
            jax.ShapeDtypeStruct((n, h), F32),
        ],
    )(x, w1a, w1b)


def _make_gather(n, h, e_slice, chunk):
    ew = e_slice // _NW
    n_chunks = ew // chunk
    mesh = plsc.VectorSubcoreMesh(core_axis_name="c", subcore_axis_name="s")

    @functools.partial(
        pl.kernel,
        out_type=jax.ShapeDtypeStruct((e_slice, h), F32),
        mesh=mesh,
        scratch_types=[
            pltpu.VMEM((ew,), jnp.int32),
            pltpu.VMEM((ew,), jnp.int32),
            pltpu.VMEM((chunk, h), F32),
            pltpu.VMEM((chunk, h), F32),
            pltpu.VMEM((chunk, h), F32),
            pltpu.VMEM((chunk, h), F32),
            pltpu.SemaphoreType.DMA,
            pltpu.SemaphoreType.DMA,
            pltpu.SemaphoreType.DMA,
            pltpu.SemaphoreType.DMA,
            pltpu.SemaphoreType.DMA,
            pltpu.SemaphoreType.DMA,
        ],
    )
    def gather_k(pa_hbm, pb_hbm, src_hbm, dst_hbm, g_hbm,
                 idx_a, idx_b, ra0, ra1, rb0, rb1,
                 sa0, sa1, sb0, sb1, sw0, sw1):
        wid = lax.axis_index("s") * _NC + lax.axis_index("c")
        base = wid * ew
        pltpu.sync_copy(src_hbm.at[pl.ds(base, ew)], idx_a)
        pltpu.sync_copy(dst_hbm.at[pl.ds(base, ew)], idx_b)
        ra = (ra0, ra1)
        rb = (rb0, rb1)
        sa = (sa0, sa1)
        sb = (sb0, sb1)
        sw = (sw0, sw1)
        cps = {}
        for i in range(n_chunks + 1):
            j = i % 2
            if i >= 2:
                cps[("w", i - 2)].wait()
            if i < n_chunks:
                ia = idx_a.at[pl.ds(i * chunk, chunk)]
                ib = idx_b.at[pl.ds(i * chunk, chunk)]
                cps[("a", i)] = pltpu.async_copy(pa_hbm.at[ia], ra[j], sa[j])
                cps[("b", i)] = pltpu.async_copy(pb_hbm.at[ib], rb[j], sb[j])
            if i >= 1:
                k = (i - 1) % 2
                cps[("a", i - 1)].wait()
                cps[("b", i - 1)].wait()
                a_buf, b_buf = ra[k], rb[k]

                def addbody(r, carry, a_buf=a_buf, b_buf=b_buf):
                    for c in range(h // 16):
                        sl = pl.ds(c * 16, 16)
                        a_buf[r, sl] = a_buf[r, sl] + b_buf[r, sl]
                    return carry

                lax.fori_loop(0, chunk, addbody, 0)
                cps[("w", i - 1)] = pltpu.async_copy(
                    a_buf, g_hbm.at[pl.ds(base + (i - 1) * chunk, chunk)],
                    sw[k])
        cps[("w", n_chunks - 1)].wait()

    return gather_k


def _make_edge_body(tb, h, row_off, grid_n, has_prev):
    def body(*refs):
        if has_prev:
            (g_ref, ea_ref, dprev_ref, w1e_ref, b1_ref, w2_ref,
             b2_ref, swr_ref, sb_ref, dnew_any, m_ref, db0, db1, ds0,
             ds1) = refs
            del dprev_ref
        else:
            (g_ref, ea_ref, w1e_ref, b1_ref, w2_ref, b2_ref,
             swr_ref, sb_ref, dnew_any, m_ref, db0, db1, ds0, ds1) = refs
        step = pl.program_id(0)
        ea = ea_ref[...]
        acc = jnp.dot(ea.astype(BF16), w1e_ref[...],
                      preferred_element_type=F32)
        hmid = jnp.maximum(acc + g_ref[...] + b1_ref[...], 0.0)
        msg = jnp.maximum(
            jnp.dot(hmid.astype(BF16), w2_ref[...],
                    preferred_element_type=F32)
            + b2_ref[...], 0.0)
        logit = jnp.sum(msg * swr_ref[...], axis=1, keepdims=True) + sb_ref[...]
        gate = 1.0 / (1.0 + jnp.exp(-logit))
        m_ref[...] = msg * gate
        dn = ea + msg

        rows = pl.ds((row_off + step) * tb, tb)
        parity = lax.rem(step, 2)
        for p, (buf, sem) in enumerate(((db0, ds0), (db1, ds1))):
            @pl.when((parity == p) & (step >= 2))
            def _(buf=buf, sem=sem):
                pltpu.make_async_copy(buf, dnew_any.at[rows], sem).wait()

            @pl.when(parity == p)
            def _(buf=buf, sem=sem):
                buf[...] = dn
                pltpu.make_async_copy(buf, dnew_any.at[rows], sem).start()

        @pl.when(step == grid_n - 1)
        def _():
            pltpu.make_async_copy(db0, dnew_any.at[rows], ds0).wait()
            pltpu.make_async_copy(db1, dnew_any.at[rows], ds1).wait()

    return body


def _edge_mlp_slice(g, ea, dprev, w1e, b1, w2, b2, swr, sb, tb,
                    slice_idx):
    e_full, h = ea.shape
    es = g.shape[0]
    grid = es // tb
    off = slice_idx * grid
    full = lambda i: (0, 0)
    tile = lambda i: (i, 0)
    shifted = lambda i: (i + off, 0)
    weight_specs = [
        pl.BlockSpec((h, h), full),
        pl.BlockSpec((1, h), full),
        pl.BlockSpec((h, h), full),
        pl.BlockSpec((1, h), full),
        pl.BlockSpec((1, h), full),
        pl.BlockSpec((1, 1), full),
    ]
    any_spec = pl.BlockSpec(memory_space=pl.ANY)
    out_specs = [
        any_spec,
        pl.BlockSpec((tb, h), tile),
    ]
    out_shape = [
        jax.ShapeDtypeStruct((e_full, h), F32),
        jax.ShapeDtypeStruct((es, h), F32),
    ]
    slice_specs = [
        pl.BlockSpec((tb, h), tile),
        pl.BlockSpec((tb, h), shifted),
    ]
    scratch = [
        pltpu.VMEM((tb, h), F32),
        pltpu.VMEM((tb, h), F32),
        pltpu.SemaphoreType.DMA,
        pltpu.SemaphoreType.DMA,
    ]
    if dprev is None:
        return pl.pallas_call(
            _make_edge_body(tb, h, off, grid, has_prev=False),
            grid=(grid,),
            in_specs=slice_specs + weight_specs,
            out_specs=out_specs,
            out_shape=out_shape,
            scratch_shapes=scratch,
        )(g, ea, w1e, b1, w2, b2, swr, sb)
    return pl.pallas_call(
        _make_edge_body(tb, h, off, grid, has_prev=True),
        grid=(grid,),
        in_specs=slice_specs + [any_spec] + weight_specs,
        out_specs=out_specs,
        out_shape=out_shape,
        scratch_shapes=scratch,
        input_output_aliases={2: 0},
    )(g, ea, dprev, w1e, b1, w2, b2, swr, sb)


def _make_scatter(n_pad, h, e_slice, n_slices, chunk):
    ew = e_slice // _NW
    n_chunks = ew // chunk
    rows_per_tile = n_pad // _NS
    mesh = plsc.VectorSubcoreMesh(core_axis_name="c", subcore_axis_name="s")

    n_tasks = n_slices * n_chunks

    @functools.partial(
        pl.kernel,
        out_type=jax.ShapeDtypeStruct((_NC, n_pad, h), F32),
        mesh=mesh,
        scratch_types=[
            pltpu.VMEM_SHARED((n_pad, h), F32),
            pltpu.VMEM((n_tasks, chunk), jnp.int32),
            pltpu.VMEM((chunk, h), F32),
            pltpu.VMEM((chunk, h), F32),
            pltpu.SemaphoreType.DMA,
            pltpu.SemaphoreType.DMA,
            pltpu.SemaphoreType.DMA,
            pltpu.SemaphoreType.DMA,
            pltpu.SemaphoreType.DMA,
            pltpu.SemaphoreType.DMA,
        ],
    )
    def scatter_k(*refs):
        m_refs = refs[:n_slices]
        dst_refs = refs[n_slices:2 * n_slices]
        part_hbm = refs[2 * n_slices]
        acc_sh, idx2d, rb0, rb1, si0, si1, sl0, sl1, ss0, ss1 = (
            refs[2 * n_slices + 1:])
        rbuf = (rb0, rb1)
        si = (si0, si1)
        sl = (sl0, sl1)
        ss = (ss0, ss1)
        cid = lax.axis_index("c")
        sid = lax.axis_index("s")
        wid = sid * _NC + cid
        base = wid * ew
        my_row0 = sid * rows_per_tile

        pieces = []
        r = 0
        while r < rows_per_tile:
            pieces.append((r, min(chunk, rows_per_tile - r)))
            r += chunk

        def zbody(i, carry):
            for j in range(h // 16):
                rb0[i, pl.ds(j * 16, 16)] = jnp.zeros((16,), F32)
            return carry

        lax.fori_loop(0, chunk, zbody, 0)
        for r0, sz in pieces:
            pltpu.sync_copy(rb0.at[pl.ds(0, sz)],
                            acc_sh.at[pl.ds(my_row0 + r0, sz)])
        plsc.subcore_barrier()

        tasks = [(s, c) for s in range(n_slices) for c in range(n_chunks)]
        cps = {}
        for t in range(n_tasks + 1):
            j = t % 2
            if t >= 2:
                cps[("s", t - 2)].wait()
            if t < n_tasks:
                s, c = tasks[t]
                off = base + c * chunk
                cps[("i", t)] = pltpu.async_copy(
                    dst_refs[s].at[pl.ds(off, chunk)], idx2d.at[t], si[j])
                cps[("m", t)] = pltpu.async_copy(
                    m_refs[s].at[pl.ds(off, chunk)], rbuf[j], sl[j])
            if t >= 1:
                k = (t - 1) % 2
                cps[("i", t - 1)].wait()
                cps[("m", t - 1)].wait()
                cps[("s", t - 1)] = pltpu.async_copy(
                    rbuf[k], acc_sh.at[idx2d.at[t - 1]], ss[k], add=True)
        cps[("s", n_tasks - 1)].wait()
        plsc.subcore_barrier()

        wcps = {}
        np_ = len(pieces)
        for t in range(np_ + 1):
            j = t % 2
            if t >= 2:
                wcps[("w", t - 2)].wait()
            if t < np_:
                r0, sz = pieces[t]
                wcps[("r", t)] = pltpu.async_copy(
                    acc_sh.at[pl.ds(my_row0 + r0, sz)],
                    rbuf[j].at[pl.ds(0, sz)], sl[j])
            if t >= 1:
                k = (t - 1) % 2
                r0, sz = pieces[t - 1]
                wcps[("r", t - 1)].wait()
                wcps[("w", t - 1)] = pltpu.async_copy(
                    rbuf[k].at[pl.ds(0, sz)],
                    part_hbm.at[cid, pl.ds(my_row0 + r0, sz)], ss[k])
        wcps[("w", np_ - 1)].wait()

    return scatter_k


def _make_update_body(n_parts):
    def _update_body(*refs):
        parts = refs[:n_parts]
        x_ref, u1_ref, ub1_ref, u2_ref, ub2_ref, out_ref = refs[n_parts:]
        x = x_ref[...]
        inp = x
        for p in parts:
            inp = inp + p[0].astype(F32)
        u = jnp.maximum(
            jnp.dot(inp, u1_ref[...], preferred_element_type=F32)
            + ub1_ref[...], 0.0)
        out_ref[...] = (
            jnp.dot(u, u2_ref[...], preferred_element_type=F32)
            + ub2_ref[...] + x)
    return _update_body


def _node_update(part_arrays, x, u1, ub1, u2, ub2, bn):
    n, h = x.shape
    grid = n // bn
    full = lambda i: (0, 0)
    tile = lambda i: (i, 0)
    part_specs = []
    part_args = []
    for arr in part_arrays:
        for c in range(_NC):
            part_specs.append(
                pl.BlockSpec((1, bn, h), lambda i, c=c: (c, i, 0)))
            part_args.append(arr)
    return pl.pallas_call(
        _make_update_body(len(part_args)),
        grid=(grid,),
        in_specs=part_specs + [
            pl.BlockSpec((bn, h), tile),
            pl.BlockSpec((h, h), full),
            pl.BlockSpec((1, h), full),
            pl.BlockSpec((h, h), full),
            pl.BlockSpec((1, h), full),
        ],
        out_specs=pl.BlockSpec((bn, h), tile),
        out_shape=jax.ShapeDtypeStruct((n, h), F32),
    )(*part_args, x, u1, ub1, u2, ub2)


def kernel(x, edge_index, edge_attr, msg_W1, msg_b1, msg_W2, msg_b2,
           soft_W, soft_b, upd_W1, upd_b1, upd_W2, upd_b2):
    n, h = x.shape
    e = edge_index.shape[1]

    src = edge_index[0]
    dst = edge_index[1]
    w1a = msg_W1[:h]
    w1b = msg_W1[h:2 * h]
    w1e = msg_W1[2 * h:]
    b1 = msg_b1.reshape(1, h)
    b2 = msg_b2.reshape(1, h)
    swr = soft_W.reshape(1, h)
    sb = soft_b.reshape(1, 1)
    ub1 = upd_b1.reshape(1, h)
    ub2 = upd_b2.reshape(1, h)

    pa, pb = _node_proj(x, w1a, w1b, bn=2000)

    n_slices = 5
    es = e // n_slices
    gather_fn = _make_gather(n, h, es, chunk=200)
    srcs = [src[i * es:(i + 1) * es] for i in range(n_slices)]
    dsts = [dst[i * es:(i + 1) * es] for i in range(n_slices)]
    gs = [gather_fn(pa, pb, srcs[i], dsts[i]) for i in range(n_slices)]

    w1e_bf = w1e.astype(BF16)
    w2_bf = msg_W2.astype(BF16)
    d_new = None
    ms = []
    for i in range(n_slices):
        d_new, m_i = _edge_mlp_slice(gs[i], edge_attr, d_new, w1e_bf,
                                     b1, w2_bf, b2, swr, sb, tb=800,
                                     slice_idx=i)
        ms.append(m_i)

    n_pad = _NS * ((n // _NS // 8 + 1) * 8)
    ga_slices, gb_slices = 4, 1
    part_a = _make_scatter(n_pad, h, es, ga_slices, chunk=80)(
        *ms[:ga_slices], *dsts[:ga_slices])
    part_b = _make_scatter(n_pad, h, es, gb_slices, chunk=80)(
        *ms[ga_slices:], *dsts[ga_slices:])
    out_feat = _node_update([part_a, part_b], x, upd_W1, ub1, upd_W2, ub2,
                            bn=2000)
    return (out_feat, d_new)


# speedup vs baseline: 4.7509x; 1.1055x over previous
import functools

import jax
import jax.numpy as jnp
from jax import lax
from jax.experimental import pallas as pl
from jax.experimental.pallas import tpu as pltpu
from jax.experimental.pallas import tpu_sc as plsc

F32 = jnp.float32
BF16 = jnp.bfloat16

_NC = 2
_NS = 16
_NW = _NC * _NS


def _proj_body(x_ref, wa_ref, wb_ref, pa_ref, pb_ref):
    x = x_ref[...]
    pa_ref[...] = jnp.dot(x, wa_ref[...], preferred_element_type=F32)
    pb_ref[...] = jnp.dot(x, wb_ref[...], preferred_element_type=F32)


def _node_proj(x, w1a, w1b, bn):
    n, h = x.shape
    grid = n // bn
    return pl.pallas_call(
        _proj_body,
        grid=(grid,),
        in_specs=[
            pl.BlockSpec((bn, h), lambda i: (i, 0)),
            pl.BlockSpec((h, h), lambda i: (0, 0)),
            pl.BlockSpec((h, h), lambda i: (0, 0)),
        ],
        out_specs=[
            pl.BlockSpec((bn, h), lambda i: (i, 0)),
            pl.BlockSpec((bn, h), lambda i: (i, 0)),
        ],
        out_shape=[
            jax.ShapeDtypeStruct((n, h), F32),
            jax.ShapeDtypeStruct((n, h), F32),
        ],
    )(x, w1a, w1b)


def _make_gather(n, h, e_slice, chunk):
    ew = e_slice // _NW
    n_chunks = ew // chunk
    mesh = plsc.VectorSubcoreMesh(core_axis_name="c", subcore_axis_name="s")

    @functools.partial(
        pl.kernel,
        out_type=jax.ShapeDtypeStruct((e_slice, h), F32),
        mesh=mesh,
        scratch_types=[
            pltpu.VMEM((ew,), jnp.int32),
            pltpu.VMEM((ew,), jnp.int32),
            pltpu.VMEM((chunk, h), F32),
            pltpu.VMEM((chunk, h), F32),
            pltpu.VMEM((chunk, h), F32),
            pltpu.VMEM((chunk, h), F32),
            pltpu.SemaphoreType.DMA,
            pltpu.SemaphoreType.DMA,
            pltpu.SemaphoreType.DMA,
            pltpu.SemaphoreType.DMA,
            pltpu.SemaphoreType.DMA,
            pltpu.SemaphoreType.DMA,
        ],
    )
    def gather_k(pa_hbm, pb_hbm, src_hbm, dst_hbm, g_hbm,
                 idx_a, idx_b, ra0, ra1, rb0, rb1,
                 sa0, sa1, sb0, sb1, sw0, sw1):
        wid = lax.axis_index("s") * _NC + lax.axis_index("c")
        base = wid * ew
        pltpu.sync_copy(src_hbm.at[pl.ds(base, ew)], idx_a)
        pltpu.sync_copy(dst_hbm.at[pl.ds(base, ew)], idx_b)
        ra = (ra0, ra1)
        rb = (rb0, rb1)
        sa = (sa0, sa1)
        sb = (sb0, sb1)
        sw = (sw0, sw1)
        cps = {}
        for i in range(n_chunks + 1):
            j = i % 2
            if i >= 2:
                cps[("w", i - 2)].wait()
            if i < n_chunks:
                ia = idx_a.at[pl.ds(i * chunk, chunk)]
                ib = idx_b.at[pl.ds(i * chunk, chunk)]
                cps[("a", i)] = pltpu.async_copy(pa_hbm.at[ia], ra[j], sa[j])
                cps[("b", i)] = pltpu.async_copy(pb_hbm.at[ib], rb[j], sb[j])
            if i >= 1:
                k = (i - 1) % 2
                cps[("a", i - 1)].wait()
                cps[("b", i - 1)].wait()
                a_buf, b_buf = ra[k], rb[k]

                def addbody(r, carry, a_buf=a_buf, b_buf=b_buf):
                    for c in range(h // 16):
                        sl = pl.ds(c * 16, 16)
                        a_buf[r, sl] = a_buf[r, sl] + b_buf[r, sl]
                    return carry

                lax.fori_loop(0, chunk, addbody, 0)
                cps[("w", i - 1)] = pltpu.async_copy(
                    a_buf, g_hbm.at[pl.ds(base + (i - 1) * chunk, chunk)],
                    sw[k])
        cps[("w", n_chunks - 1)].wait()

    return gather_k


def _make_edge_body(tb, h, row_off, grid_n, has_prev):
    def body(*refs):
        if has_prev:
            (g_ref, ea_ref, dprev_ref, w1e_ref, b1_ref, w2_ref,
             b2_ref, swr_ref, sb_ref, dnew_any, m_ref, db0, db1, ds0,
             ds1) = refs
            del dprev_ref
        else:
            (g_ref, ea_ref, w1e_ref, b1_ref, w2_ref, b2_ref,
             swr_ref, sb_ref, dnew_any, m_ref, db0, db1, ds0, ds1) = refs
        step = pl.program_id(0)
        ea = ea_ref[...]
        acc = jnp.dot(ea.astype(BF16), w1e_ref[...],
                      preferred_element_type=F32)
        hmid = jnp.maximum(acc + g_ref[...] + b1_ref[...], 0.0)
        msg = jnp.maximum(
            jnp.dot(hmid.astype(BF16), w2_ref[...],
                    preferred_element_type=F32)
            + b2_ref[...], 0.0)
        logit = jnp.sum(msg * swr_ref[...], axis=1, keepdims=True) + sb_ref[...]
        gate = 1.0 / (1.0 + jnp.exp(-logit))
        m_ref[...] = msg * gate
        dn = ea + msg

        rows = pl.ds((row_off + step) * tb, tb)
        parity = lax.rem(step, 2)
        for p, (buf, sem) in enumerate(((db0, ds0), (db1, ds1))):
            @pl.when((parity == p) & (step >= 2))
            def _(buf=buf, sem=sem):
                pltpu.make_async_copy(buf, dnew_any.at[rows], sem).wait()

            @pl.when(parity == p)
            def _(buf=buf, sem=sem):
                buf[...] = dn
                pltpu.make_async_copy(buf, dnew_any.at[rows], sem).start()

        @pl.when(step == grid_n - 1)
        def _():
            pltpu.make_async_copy(db0, dnew_any.at[rows], ds0).wait()
            pltpu.make_async_copy(db1, dnew_any.at[rows], ds1).wait()

    return body


def _edge_mlp_slice(g, ea, dprev, w1e, b1, w2, b2, swr, sb, tb,
                    slice_idx):
    e_full, h = ea.shape
    es = g.shape[0]
    grid = es // tb
    off = slice_idx * grid
    full = lambda i: (0, 0)
    tile = lambda i: (i, 0)
    shifted = lambda i: (i + off, 0)
    weight_specs = [
        pl.BlockSpec((h, h), full),
        pl.BlockSpec((1, h), full),
        pl.BlockSpec((h, h), full),
        pl.BlockSpec((1, h), full),
        pl.BlockSpec((1, h), full),
        pl.BlockSpec((1, 1), full),
    ]
    any_spec = pl.BlockSpec(memory_space=pl.ANY)
    out_specs = [
        any_spec,
        pl.BlockSpec((tb, h), tile),
    ]
    out_shape = [
        jax.ShapeDtypeStruct((e_full, h), F32),
        jax.ShapeDtypeStruct((es, h), F32),
    ]
    slice_specs = [
        pl.BlockSpec((tb, h), tile),
        pl.BlockSpec((tb, h), shifted),
    ]
    scratch = [
        pltpu.VMEM((tb, h), F32),
        pltpu.VMEM((tb, h), F32),
        pltpu.SemaphoreType.DMA,
        pltpu.SemaphoreType.DMA,
    ]
    if dprev is None:
        return pl.pallas_call(
            _make_edge_body(tb, h, off, grid, has_prev=False),
            grid=(grid,),
            in_specs=slice_specs + weight_specs,
            out_specs=out_specs,
            out_shape=out_shape,
            scratch_shapes=scratch,
        )(g, ea, w1e, b1, w2, b2, swr, sb)
    return pl.pallas_call(
        _make_edge_body(tb, h, off, grid, has_prev=True),
        grid=(grid,),
        in_specs=slice_specs + [any_spec] + weight_specs,
        out_specs=out_specs,
        out_shape=out_shape,
        scratch_shapes=scratch,
        input_output_aliases={2: 0},
    )(g, ea, dprev, w1e, b1, w2, b2, swr, sb)


def _make_scatter(n_pad, h, e_slice, n_slices, chunk):
    ew = e_slice // _NW
    n_chunks = ew // chunk
    rows_per_tile = n_pad // _NS
    mesh = plsc.VectorSubcoreMesh(core_axis_name="c", subcore_axis_name="s")

    n_tasks = n_slices * n_chunks

    @functools.partial(
        pl.kernel,
        out_type=jax.ShapeDtypeStruct((_NC, n_pad, h), F32),
        mesh=mesh,
        scratch_types=[
            pltpu.VMEM_SHARED((n_pad, h), F32),
            pltpu.VMEM((n_tasks, chunk), jnp.int32),
            pltpu.VMEM((chunk, h), F32),
            pltpu.VMEM((chunk, h), F32),
            pltpu.SemaphoreType.DMA,
            pltpu.SemaphoreType.DMA,
            pltpu.SemaphoreType.DMA,
            pltpu.SemaphoreType.DMA,
            pltpu.SemaphoreType.DMA,
            pltpu.SemaphoreType.DMA,
        ],
    )
    def scatter_k(*refs):
        m_refs = refs[:n_slices]
        dst_refs = refs[n_slices:2 * n_slices]
        part_hbm = refs[2 * n_slices]
        acc_sh, idx2d, rb0, rb1, si0, si1, sl0, sl1, ss0, ss1 = (
            refs[2 * n_slices + 1:])
        rbuf = (rb0, rb1)
        si = (si0, si1)
        sl = (sl0, sl1)
        ss = (ss0, ss1)
        cid = lax.axis_index("c")
        sid = lax.axis_index("s")
        wid = sid * _NC + cid
        base = wid * ew
        my_row0 = sid * rows_per_tile

        pieces = []
        r = 0
        while r < rows_per_tile:
            pieces.append((r, min(chunk, rows_per_tile - r)))
            r += chunk

        def zbody(i, carry):
            for j in range(h // 16):
                rb0[i, pl.ds(j * 16, 16)] = jnp.zeros((16,), F32)
            return carry

        lax.fori_loop(0, chunk, zbody, 0)
        for r0, sz in pieces:
            pltpu.sync_copy(rb0.at[pl.ds(0, sz)],
                            acc_sh.at[pl.ds(my_row0 + r0, sz)])
        plsc.subcore_barrier()

        tasks = [(s, c) for s in range(n_slices) for c in range(n_chunks)]
        cps = {}
        for t in range(n_tasks + 1):
            j = t % 2
            if t >= 2:
                cps[("s", t - 2)].wait()
            if t < n_tasks:
                s, c = tasks[t]
                off = base + c * chunk
                cps[("i", t)] = pltpu.async_copy(
                    dst_refs[s].at[pl.ds(off, chunk)], idx2d.at[t], si[j])
                cps[("m", t)] = pltpu.async_copy(
                    m_refs[s].at[pl.ds(off, chunk)], rbuf[j], sl[j])
            if t >= 1:
                k = (t - 1) % 2
                cps[("i", t - 1)].wait()
                cps[("m", t - 1)].wait()
                cps[("s", t - 1)] = pltpu.async_copy(
                    rbuf[k], acc_sh.at[idx2d.at[t - 1]], ss[k], add=True)
        cps[("s", n_tasks - 1)].wait()
        plsc.subcore_barrier()

        wcps = {}
        np_ = len(pieces)
        for t in range(np_ + 1):
            j = t % 2
            if t >= 2:
                wcps[("w", t - 2)].wait()
            if t < np_:
                r0, sz = pieces[t]
                wcps[("r", t)] = pltpu.async_copy(
                    acc_sh.at[pl.ds(my_row0 + r0, sz)],
                    rbuf[j].at[pl.ds(0, sz)], sl[j])
            if t >= 1:
                k = (t - 1) % 2
                r0, sz = pieces[t - 1]
                wcps[("r", t - 1)].wait()
                wcps[("w", t - 1)] = pltpu.async_copy(
                    rbuf[k].at[pl.ds(0, sz)],
                    part_hbm.at[cid, pl.ds(my_row0 + r0, sz)], ss[k])
        wcps[("w", np_ - 1)].wait()

    return scatter_k


def _make_update_body(n_parts):
    def _update_body(*refs):
        parts = refs[:n_parts]
        x_ref, u1_ref, ub1_ref, u2_ref, ub2_ref, out_ref = refs[n_parts:]
        x = x_ref[...]
        inp = x
        for p in parts:
            inp = inp + p[0].astype(F32)
        u = jnp.maximum(
            jnp.dot(inp, u1_ref[...], preferred_element_type=F32)
            + ub1_ref[...], 0.0)
        out_ref[...] = (
            jnp.dot(u, u2_ref[...], preferred_element_type=F32)
            + ub2_ref[...] + x)
    return _update_body


def _node_update(part_arrays, x, u1, ub1, u2, ub2, bn):
    n, h = x.shape
    grid = n // bn
    full = lambda i: (0, 0)
    tile = lambda i: (i, 0)
    part_specs = []
    part_args = []
    for arr in part_arrays:
        for c in range(_NC):
            part_specs.append(
                pl.BlockSpec((1, bn, h), lambda i, c=c: (c, i, 0)))
            part_args.append(arr)
    return pl.pallas_call(
        _make_update_body(len(part_args)),
        grid=(grid,),
        in_specs=part_specs + [
            pl.BlockSpec((bn, h), tile),
            pl.BlockSpec((h, h), full),
            pl.BlockSpec((1, h), full),
            pl.BlockSpec((h, h), full),
            pl.BlockSpec((1, h), full),
        ],
        out_specs=pl.BlockSpec((bn, h), tile),
        out_shape=jax.ShapeDtypeStruct((n, h), F32),
    )(*part_args, x, u1, ub1, u2, ub2)


def kernel(x, edge_index, edge_attr, msg_W1, msg_b1, msg_W2, msg_b2,
           soft_W, soft_b, upd_W1, upd_b1, upd_W2, upd_b2):
    n, h = x.shape
    e = edge_index.shape[1]

    src = edge_index[0]
    dst = edge_index[1]
    w1a = msg_W1[:h]
    w1b = msg_W1[h:2 * h]
    w1e = msg_W1[2 * h:]
    b1 = msg_b1.reshape(1, h)
    b2 = msg_b2.reshape(1, h)
    swr = soft_W.reshape(1, h)
    sb = soft_b.reshape(1, 1)
    ub1 = upd_b1.reshape(1, h)
    ub2 = upd_b2.reshape(1, h)

    pa, pb = _node_proj(x, w1a, w1b, bn=2000)

    n_slices = 5
    es = e // n_slices
    gather_fn = _make_gather(n, h, es, chunk=200)
    srcs = [src[i * es:(i + 1) * es] for i in range(n_slices)]
    dsts = [dst[i * es:(i + 1) * es] for i in range(n_slices)]
    gs = [gather_fn(pa, pb, srcs[i], dsts[i]) for i in range(n_slices)]

    w1e_bf = w1e.astype(BF16)
    w2_bf = msg_W2.astype(BF16)
    d_new = None
    ms = []
    for i in range(n_slices):
        d_new, m_i = _edge_mlp_slice(gs[i], edge_attr, d_new, w1e_bf,
                                     b1, w2_bf, b2, swr, sb, tb=1600,
                                     slice_idx=i)
        ms.append(m_i)

    n_pad = _NS * ((n // _NS // 8 + 1) * 8)
    ga_slices, gb_slices = 4, 1
    part_a = _make_scatter(n_pad, h, es, ga_slices, chunk=80)(
        *ms[:ga_slices], *dsts[:ga_slices])
    part_b = _make_scatter(n_pad, h, es, gb_slices, chunk=80)(
        *ms[ga_slices:], *dsts[ga_slices:])
    out_feat = _node_update([part_a, part_b], x, upd_W1, ub1, upd_W2, ub2,
                            bn=2000)
    return (out_feat, d_new)


# speedup vs baseline: 5.0835x; 1.0700x over previous
import functools

import jax
import jax.numpy as jnp
from jax import lax
from jax.experimental import pallas as pl
from jax.experimental.pallas import tpu as pltpu
from jax.experimental.pallas import tpu_sc as plsc

F32 = jnp.float32
BF16 = jnp.bfloat16

_NC = 2
_NS = 16
_NW = _NC * _NS


def _proj_body(x_ref, wa_ref, wb_ref, pa_ref, pb_ref):
    x = x_ref[...]
    pa_ref[...] = jnp.dot(x, wa_ref[...], preferred_element_type=F32)
    pb_ref[...] = jnp.dot(x, wb_ref[...], preferred_element_type=F32)


def _node_proj(x, w1a, w1b, bn):
    n, h = x.shape
    grid = n // bn
    return pl.pallas_call(
        _proj_body,
        grid=(grid,),
        in_specs=[
            pl.BlockSpec((bn, h), lambda i: (i, 0)),
            pl.BlockSpec((h, h), lambda i: (0, 0)),
            pl.BlockSpec((h, h), lambda i: (0, 0)),
        ],
        out_specs=[
            pl.BlockSpec((bn, h), lambda i: (i, 0)),
            pl.BlockSpec((bn, h), lambda i: (i, 0)),
        ],
        out_shape=[
            jax.ShapeDtypeStruct((n, h), F32),
            jax.ShapeDtypeStruct((n, h), F32),
        ],
    )(x, w1a, w1b)


def _make_gather(n, h, e_slice, chunk):
    ew = e_slice // _NW
    n_chunks = ew // chunk
    mesh = plsc.VectorSubcoreMesh(core_axis_name="c", subcore_axis_name="s")

    @functools.partial(
        pl.kernel,
        out_type=jax.ShapeDtypeStruct((e_slice, h), F32),
        mesh=mesh,
        scratch_types=[
            pltpu.VMEM((ew,), jnp.int32),
            pltpu.VMEM((ew,), jnp.int32),
            pltpu.VMEM((chunk, h), F32),
            pltpu.VMEM((chunk, h), F32),
            pltpu.VMEM((chunk, h), F32),
            pltpu.VMEM((chunk, h), F32),
            pltpu.SemaphoreType.DMA,
            pltpu.SemaphoreType.DMA,
            pltpu.SemaphoreType.DMA,
            pltpu.SemaphoreType.DMA,
            pltpu.SemaphoreType.DMA,
            pltpu.SemaphoreType.DMA,
        ],
    )
    def gather_k(pa_hbm, pb_hbm, src_hbm, dst_hbm, g_hbm,
                 idx_a, idx_b, ra0, ra1, rb0, rb1,
                 sa0, sa1, sb0, sb1, sw0, sw1):
        wid = lax.axis_index("s") * _NC + lax.axis_index("c")
        base = wid * ew
        pltpu.sync_copy(src_hbm.at[pl.ds(base, ew)], idx_a)
        pltpu.sync_copy(dst_hbm.at[pl.ds(base, ew)], idx_b)
        ra = (ra0, ra1)
        rb = (rb0, rb1)
        sa = (sa0, sa1)
        sb = (sb0, sb1)
        sw = (sw0, sw1)
        cps = {}
        for i in range(n_chunks + 1):
            j = i % 2
            if i >= 2:
                cps[("w", i - 2)].wait()
            if i < n_chunks:
                ia = idx_a.at[pl.ds(i * chunk, chunk)]
                ib = idx_b.at[pl.ds(i * chunk, chunk)]
                cps[("a", i)] = pltpu.async_copy(pa_hbm.at[ia], ra[j], sa[j])
                cps[("b", i)] = pltpu.async_copy(pb_hbm.at[ib], rb[j], sb[j])
            if i >= 1:
                k = (i - 1) % 2
                cps[("a", i - 1)].wait()
                cps[("b", i - 1)].wait()
                a_buf, b_buf = ra[k], rb[k]

                def addbody(r, carry, a_buf=a_buf, b_buf=b_buf):
                    for c in range(h // 16):
                        sl = pl.ds(c * 16, 16)
                        a_buf[r, sl] = a_buf[r, sl] + b_buf[r, sl]
                    return carry

                lax.fori_loop(0, chunk, addbody, 0)
                cps[("w", i - 1)] = pltpu.async_copy(
                    a_buf, g_hbm.at[pl.ds(base + (i - 1) * chunk, chunk)],
                    sw[k])
        cps[("w", n_chunks - 1)].wait()

    return gather_k


def _make_edge_body(tb, h, row_off, grid_n, has_prev):
    def body(*refs):
        if has_prev:
            (g_ref, ea_ref, dprev_ref, w1e_ref, b1_ref, w2_ref,
             b2_ref, swr_ref, sb_ref, dnew_any, m_ref, db0, db1, ds0,
             ds1) = refs
            del dprev_ref
        else:
            (g_ref, ea_ref, w1e_ref, b1_ref, w2_ref, b2_ref,
             swr_ref, sb_ref, dnew_any, m_ref, db0, db1, ds0, ds1) = refs
        step = pl.program_id(0)
        ea = ea_ref[...]
        acc = jnp.dot(ea.astype(BF16), w1e_ref[...],
                      preferred_element_type=F32)
        hmid = jnp.maximum(acc + g_ref[...] + b1_ref[...], 0.0)
        msg = jnp.maximum(
            jnp.dot(hmid.astype(BF16), w2_ref[...],
                    preferred_element_type=F32)
            + b2_ref[...], 0.0)
        logit = jnp.sum(msg * swr_ref[...], axis=1, keepdims=True) + sb_ref[...]
        gate = 1.0 / (1.0 + jnp.exp(-logit))
        m_ref[...] = msg * gate
        dn = ea + msg

        rows = pl.ds((row_off + step) * tb, tb)
        parity = lax.rem(step, 2)
        for p, (buf, sem) in enumerate(((db0, ds0), (db1, ds1))):
            @pl.when((parity == p) & (step >= 2))
            def _(buf=buf, sem=sem):
                pltpu.make_async_copy(buf, dnew_any.at[rows], sem).wait()

            @pl.when(parity == p)
            def _(buf=buf, sem=sem):
                buf[...] = dn
                pltpu.make_async_copy(buf, dnew_any.at[rows], sem).start()

        @pl.when(step == grid_n - 1)
        def _():
            pltpu.make_async_copy(db0, dnew_any.at[rows], ds0).wait()
            pltpu.make_async_copy(db1, dnew_any.at[rows], ds1).wait()

    return body


def _edge_mlp_slice(g, ea, dprev, w1e, b1, w2, b2, swr, sb, tb,
                    slice_idx):
    e_full, h = ea.shape
    es = g.shape[0]
    grid = es // tb
    off = slice_idx * grid
    full = lambda i: (0, 0)
    tile = lambda i: (i, 0)
    shifted = lambda i: (i + off, 0)
    weight_specs = [
        pl.BlockSpec((h, h), full),
        pl.BlockSpec((1, h), full),
        pl.BlockSpec((h, h), full),
        pl.BlockSpec((1, h), full),
        pl.BlockSpec((1, h), full),
        pl.BlockSpec((1, 1), full),
    ]
    any_spec = pl.BlockSpec(memory_space=pl.ANY)
    out_specs = [
        any_spec,
        pl.BlockSpec((tb, h), tile),
    ]
    out_shape = [
        jax.ShapeDtypeStruct((e_full, h), F32),
        jax.ShapeDtypeStruct((es, h), F32),
    ]
    slice_specs = [
        pl.BlockSpec((tb, h), tile),
        pl.BlockSpec((tb, h), shifted),
    ]
    scratch = [
        pltpu.VMEM((tb, h), F32),
        pltpu.VMEM((tb, h), F32),
        pltpu.SemaphoreType.DMA,
        pltpu.SemaphoreType.DMA,
    ]
    if dprev is None:
        return pl.pallas_call(
            _make_edge_body(tb, h, off, grid, has_prev=False),
            grid=(grid,),
            in_specs=slice_specs + weight_specs,
            out_specs=out_specs,
            out_shape=out_shape,
            scratch_shapes=scratch,
        )(g, ea, w1e, b1, w2, b2, swr, sb)
    return pl.pallas_call(
        _make_edge_body(tb, h, off, grid, has_prev=True),
        grid=(grid,),
        in_specs=slice_specs + [any_spec] + weight_specs,
        out_specs=out_specs,
        out_shape=out_shape,
        scratch_shapes=scratch,
        input_output_aliases={2: 0},
    )(g, ea, dprev, w1e, b1, w2, b2, swr, sb)


def _make_scatter(n_pad, h, e_slice, n_slices, chunk):
    ew = e_slice // _NW
    n_chunks = ew // chunk
    rows_per_tile = n_pad // _NS
    mesh = plsc.VectorSubcoreMesh(core_axis_name="c", subcore_axis_name="s")

    n_tasks = n_slices * n_chunks

    @functools.partial(
        pl.kernel,
        out_type=jax.ShapeDtypeStruct((_NC, n_pad, h), F32),
        mesh=mesh,
        scratch_types=[
            pltpu.VMEM_SHARED((n_pad, h), F32),
            pltpu.VMEM((n_tasks, chunk), jnp.int32),
            pltpu.VMEM((chunk, h), F32),
            pltpu.VMEM((chunk, h), F32),
            pltpu.SemaphoreType.DMA,
            pltpu.SemaphoreType.DMA,
            pltpu.SemaphoreType.DMA,
            pltpu.SemaphoreType.DMA,
            pltpu.SemaphoreType.DMA,
            pltpu.SemaphoreType.DMA,
        ],
    )
    def scatter_k(*refs):
        m_refs = refs[:n_slices]
        dst_refs = refs[n_slices:2 * n_slices]
        part_hbm = refs[2 * n_slices]
        acc_sh, idx2d, rb0, rb1, si0, si1, sl0, sl1, ss0, ss1 = (
            refs[2 * n_slices + 1:])
        rbuf = (rb0, rb1)
        si = (si0, si1)
        sl = (sl0, sl1)
        ss = (ss0, ss1)
        cid = lax.axis_index("c")
        sid = lax.axis_index("s")
        wid = sid * _NC + cid
        base = wid * ew
        my_row0 = sid * rows_per_tile

        pieces = []
        r = 0
        while r < rows_per_tile:
            pieces.append((r, min(chunk, rows_per_tile - r)))
            r += chunk

        def zbody(i, carry):
            for j in range(h // 16):
                rb0[i, pl.ds(j * 16, 16)] = jnp.zeros((16,), F32)
            return carry

        lax.fori_loop(0, chunk, zbody, 0)
        for r0, sz in pieces:
            pltpu.sync_copy(rb0.at[pl.ds(0, sz)],
                            acc_sh.at[pl.ds(my_row0 + r0, sz)])
        plsc.subcore_barrier()

        tasks = [(s, c) for s in range(n_slices) for c in range(n_chunks)]
        cps = {}
        for t in range(n_tasks + 1):
            j = t % 2
            if t >= 2:
                cps[("s", t - 2)].wait()
            if t < n_tasks:
                s, c = tasks[t]
                off = base + c * chunk
                cps[("i", t)] = pltpu.async_copy(
                    dst_refs[s].at[pl.ds(off, chunk)], idx2d.at[t], si[j])
                cps[("m", t)] = pltpu.async_copy(
                    m_refs[s].at[pl.ds(off, chunk)], rbuf[j], sl[j])
            if t >= 1:
                k = (t - 1) % 2
                cps[("i", t - 1)].wait()
                cps[("m", t - 1)].wait()
                cps[("s", t - 1)] = pltpu.async_copy(
                    rbuf[k], acc_sh.at[idx2d.at[t - 1]], ss[k], add=True)
        cps[("s", n_tasks - 1)].wait()
        plsc.subcore_barrier()

        wcps = {}
        np_ = len(pieces)
        for t in range(np_ + 1):
            j = t % 2
            if t >= 2:
                wcps[("w", t - 2)].wait()
            if t < np_:
                r0, sz = pieces[t]
                wcps[("r", t)] = pltpu.async_copy(
                    acc_sh.at[pl.ds(my_row0 + r0, sz)],
                    rbuf[j].at[pl.ds(0, sz)], sl[j])
            if t >= 1:
                k = (t - 1) % 2
                r0, sz = pieces[t - 1]
                wcps[("r", t - 1)].wait()
                wcps[("w", t - 1)] = pltpu.async_copy(
                    rbuf[k].at[pl.ds(0, sz)],
                    part_hbm.at[cid, pl.ds(my_row0 + r0, sz)], ss[k])
        wcps[("w", np_ - 1)].wait()

    return scatter_k


def _make_update_body(n_parts):
    def _update_body(*refs):
        parts = refs[:n_parts]
        x_ref, u1_ref, ub1_ref, u2_ref, ub2_ref, out_ref = refs[n_parts:]
        x = x_ref[...]
        inp = x
        for p in parts:
            inp = inp + p[0].astype(F32)
        u = jnp.maximum(
            jnp.dot(inp, u1_ref[...], preferred_element_type=F32)
            + ub1_ref[...], 0.0)
        out_ref[...] = (
            jnp.dot(u, u2_ref[...], preferred_element_type=F32)
            + ub2_ref[...] + x)
    return _update_body


def _node_update(part_arrays, x, u1, ub1, u2, ub2, bn):
    n, h = x.shape
    grid = n // bn
    full = lambda i: (0, 0)
    tile = lambda i: (i, 0)
    part_specs = []
    part_args = []
    for arr in part_arrays:
        for c in range(_NC):
            part_specs.append(
                pl.BlockSpec((1, bn, h), lambda i, c=c: (c, i, 0)))
            part_args.append(arr)
    return pl.pallas_call(
        _make_update_body(len(part_args)),
        grid=(grid,),
        in_specs=part_specs + [
            pl.BlockSpec((bn, h), tile),
            pl.BlockSpec((h, h), full),
            pl.BlockSpec((1, h), full),
            pl.BlockSpec((h, h), full),
            pl.BlockSpec((1, h), full),
        ],
        out_specs=pl.BlockSpec((bn, h), tile),
        out_shape=jax.ShapeDtypeStruct((n, h), F32),
    )(*part_args, x, u1, ub1, u2, ub2)


def kernel(x, edge_index, edge_attr, msg_W1, msg_b1, msg_W2, msg_b2,
           soft_W, soft_b, upd_W1, upd_b1, upd_W2, upd_b2):
    n, h = x.shape
    e = edge_index.shape[1]

    src = edge_index[0]
    dst = edge_index[1]
    w1a = msg_W1[:h]
    w1b = msg_W1[h:2 * h]
    w1e = msg_W1[2 * h:]
    b1 = msg_b1.reshape(1, h)
    b2 = msg_b2.reshape(1, h)
    swr = soft_W.reshape(1, h)
    sb = soft_b.reshape(1, 1)
    ub1 = upd_b1.reshape(1, h)
    ub2 = upd_b2.reshape(1, h)

    pa, pb = _node_proj(x, w1a, w1b, bn=2000)

    n_slices = 5
    es = e // n_slices
    gather_fn = _make_gather(n, h, es, chunk=200)
    srcs = [src[i * es:(i + 1) * es] for i in range(n_slices)]
    dsts = [dst[i * es:(i + 1) * es] for i in range(n_slices)]
    gs = [gather_fn(pa, pb, srcs[i], dsts[i]) for i in range(n_slices)]

    w1e_bf = w1e.astype(BF16)
    w2_bf = msg_W2.astype(BF16)
    d_new = None
    ms = []
    for i in range(n_slices):
        d_new, m_i = _edge_mlp_slice(gs[i], edge_attr, d_new, w1e_bf,
                                     b1, w2_bf, b2, swr, sb, tb=3200,
                                     slice_idx=i)
        ms.append(m_i)

    n_pad = _NS * ((n // _NS // 8 + 1) * 8)
    ga_slices, gb_slices = 4, 1
    part_a = _make_scatter(n_pad, h, es, ga_slices, chunk=80)(
        *ms[:ga_slices], *dsts[:ga_slices])
    part_b = _make_scatter(n_pad, h, es, gb_slices, chunk=80)(
        *ms[ga_slices:], *dsts[ga_slices:])
    out_feat = _node_update([part_a, part_b], x, upd_W1, ub1, upd_W2, ub2,
                            bn=2000)
    return (out_feat, d_new)
